# baseline (speedup 1.0000x reference)
import functools

import jax
import jax.numpy as jnp
from jax import lax
from jax.experimental import pallas as pl
from jax.experimental.pallas import tpu as pltpu
from jax.experimental.pallas import tpu_sc as plsc

B, D, N, K = 8, 64, 4096, 16
BNT = B * N
BT = 512
NB = N // BT
NW = 32
P = BNT // NW
G = 32
GK = G * K
NCH = P // G
NH = NCH // 2
L = 16
EPS = 1e-5


def _gelu(v):
    a1, a2, a3, a4, a5 = (0.254829592, -0.284496736, 1.421413741,
                          -1.453152027, 1.061405429)
    p = 0.3275911
    u = v * 0.7071067811865476
    s = jnp.sign(u)
    ua = jnp.abs(u)
    t = 1.0 / (1.0 + p * ua)
    poly = ((((a5 * t + a4) * t + a3) * t + a2) * t + a1) * t
    erf = s * (1.0 - poly * jnp.exp(-ua * ua))
    return 0.5 * v * (1.0 + erf)


def _mm_in_body(x_ref, wy_ref, wz_ref, y_ref, z_ref):
    xb = x_ref[0]
    dn = (((0,), (0,)), ((), ()))
    y_ref[...] = lax.dot_general(
        xb, wy_ref[...], dn,
        preferred_element_type=jnp.float32).astype(jnp.bfloat16)
    z_ref[...] = lax.dot_general(xb, wz_ref[...], dn,
                                 preferred_element_type=jnp.float32)


def _mm_in(x, wy, wz):
    return pl.pallas_call(
        _mm_in_body,
        grid=(B, NB),
        in_specs=[
            pl.BlockSpec((1, D, BT), lambda b, j: (b, 0, j)),
            pl.BlockSpec((D, D), lambda b, j: (0, 0)),
            pl.BlockSpec((D, D), lambda b, j: (0, 0)),
        ],
        out_specs=[pl.BlockSpec((BT, D), lambda b, j: (b * NB + j, 0))] * 2,
        out_shape=[jax.ShapeDtypeStruct((BNT, D), jnp.bfloat16),
                   jax.ShapeDtypeStruct((BNT, D), jnp.float32)],
    )(x, wy, wz)


def _bn_coeffs(ps, pq, g, bt):
    ssum = jnp.sum(ps, axis=0)
    ssq = jnp.sum(pq, axis=0)
    mean = ssum * (1.0 / BNT)
    var = ssq * (1.0 / BNT) - mean * mean
    scale = g[0] * lax.rsqrt(var + EPS)
    shift = bt[0] - mean * scale
    return scale, shift


def _mm_mid_body(t_ref, ps_ref, pq_ref, g_ref, b_ref, wy_ref, wz_ref, x_ref,
                 y_ref, z_ref):
    scale, shift = _bn_coeffs(ps_ref[...], pq_ref[...], g_ref[...], b_ref[...])
    h = _gelu(t_ref[...] * scale[None, :] + shift[None, :])
    dn = (((1,), (0,)), ((), ()))
    y_ref[...] = lax.dot_general(
        h, wy_ref[...], dn,
        preferred_element_type=jnp.float32).astype(jnp.bfloat16)
    z_ref[...] = lax.dot_general(h, wz_ref[...], dn,
                                 preferred_element_type=jnp.float32) \
        + jnp.transpose(x_ref[0])


def _mm_mid(t1, ps, pq, g, bt, wy, wz, x):
    return pl.pallas_call(
        _mm_mid_body,
        grid=(B, NB),
        in_specs=[
            pl.BlockSpec((BT, D), lambda b, j: (b * NB + j, 0)),
            pl.BlockSpec((NW, D), lambda b, j: (0, 0)),
            pl.BlockSpec((NW, D), lambda b, j: (0, 0)),
            pl.BlockSpec((1, D), lambda b, j: (0, 0)),
            pl.BlockSpec((1, D), lambda b, j: (0, 0)),
            pl.BlockSpec((D, D), lambda b, j: (0, 0)),
            pl.BlockSpec((D, D), lambda b, j: (0, 0)),
            pl.BlockSpec((1, D, BT), lambda b, j: (b, 0, j)),
        ],
        out_specs=[pl.BlockSpec((BT, D), lambda b, j: (b * NB + j, 0))] * 2,
        out_shape=[jax.ShapeDtypeStruct((BNT, D), jnp.bfloat16),
                   jax.ShapeDtypeStruct((BNT, D), jnp.float32)],
    )(t1, ps, pq, g, bt, wy, wz, x)


def _mm_out_body(t_ref, ps_ref, pq_ref, g_ref, b_ref, out_ref):
    scale, shift = _bn_coeffs(ps_ref[...], pq_ref[...], g_ref[...], b_ref[...])
    r = _gelu(t_ref[...] * scale[None, :] + shift[None, :])
    out_ref[0] = jnp.transpose(r)


def _mm_out(t2, ps, pq, g, bt):
    return pl.pallas_call(
        _mm_out_body,
        grid=(B, NB),
        in_specs=[
            pl.BlockSpec((BT, D), lambda b, j: (b * NB + j, 0)),
            pl.BlockSpec((NW, D), lambda b, j: (0, 0)),
            pl.BlockSpec((NW, D), lambda b, j: (0, 0)),
            pl.BlockSpec((1, D), lambda b, j: (0, 0)),
            pl.BlockSpec((1, D), lambda b, j: (0, 0)),
        ],
        out_specs=pl.BlockSpec((1, D, BT), lambda b, j: (b, 0, j)),
        out_shape=jax.ShapeDtypeStruct((B, D, N), jnp.float32),
    )(t2, ps, pq, g, bt)


def _sc_gather_max_body(y_hbm, z_hbm, gidx_hbm, t_hbm, pss_hbm, psq_hbm,
                        idx_v, ysh, rows0, rows1, z0, z1, t0, t1,
                        accs_v, accq_v, sg0, sg1, sz0, sz1, sw0, sw1):
    cid = lax.axis_index("c")
    sid = lax.axis_index("s")
    wid = cid * 16 + sid
    base = wid * P

    hb = BNT // 2
    st = hb // 16
    nst = st // GK
    for ss in range(nst):
        lo = sid * st + ss * GK
        pltpu.sync_copy(y_hbm.at[pl.ds(cid * hb + lo, GK)], rows0)
        pltpu.sync_copy(rows0, ysh.at[pl.ds(lo, GK)])
    pltpu.sync_copy(gidx_hbm.at[pl.ds(base * K, P * K)], idx_v)
    plsc.subcore_barrier()

    def g_copy(c, rows_v, sem):
        return pltpu.make_async_copy(
            ysh.at[idx_v.at[pl.ds(c * GK, GK)]], rows_v, sem)

    def z_copy(c, z_v, sem):
        return pltpu.make_async_copy(z_hbm.at[pl.ds(base + c * G, G)], z_v, sem)

    def w_copy(c, t_v, sem):
        return pltpu.make_async_copy(t_v, t_hbm.at[pl.ds(base + c * G, G)], sem)

    def compute(rows_v, z_v, t_v, accs):
        new = list(accs)
        for i in range(G):
            for j2 in range(D // (2 * L)):
                sl = pl.ds(2 * L * j2, 2 * L)
                m = rows_v[i * K, sl]
                for kk in range(1, K):
                    m = jnp.maximum(m, rows_v[i * K + kk, sl])
                ga, gb = plsc.unpack(m, format=plsc.PackFormat.INTERLEAVED)
                for j, gv in ((2 * j2, ga), (2 * j2 + 1, gb)):
                    sj = pl.ds(L * j, L)
                    t = gv + z_v[i, sj]
                    t_v[i, sj] = t
                    new[j] = new[j] + t
                    new[4 + j] = new[4 + j] + t * t
        return tuple(new)

    g_copy(0, rows0, sg0).start()
    z_copy(0, z0, sz0).start()

    zero = jnp.zeros((L,), jnp.float32)

    def body(s, accs):
        c0 = 2 * s
        c1 = c0 + 1
        g_copy(c1, rows1, sg1).start()
        z_copy(c1, z1, sz1).start()
        g_copy(c0, rows0, sg0).wait()
        z_copy(c0, z0, sz0).wait()

        @pl.when(s > 0)
        def _():
            w_copy(c0 - 2, t0, sw0).wait()

        accs = compute(rows0, z0, t0, accs)
        w_copy(c0, t0, sw0).start()

        @pl.when(s + 1 < NH)
        def _():
            g_copy(c0 + 2, rows0, sg0).start()
            z_copy(c0 + 2, z0, sz0).start()

        g_copy(c1, rows1, sg1).wait()
        z_copy(c1, z1, sz1).wait()

        @pl.when(s > 0)
        def _():
            w_copy(c1 - 2, t1, sw1).wait()

        accs = compute(rows1, z1, t1, accs)
        w_copy(c1, t1, sw1).start()
        return accs

    accs = lax.fori_loop(0, NH, body, tuple(zero for _ in range(8)))
    w_copy(NCH - 2, t0, sw0).wait()
    w_copy(NCH - 1, t1, sw1).wait()
    for j in range(D // L):
        accs_v[pl.ds(L * j, L)] = accs[j]
        accq_v[pl.ds(L * j, L)] = accs[4 + j]
    pltpu.sync_copy(accs_v, pss_hbm.at[wid])
    pltpu.sync_copy(accq_v, psq_hbm.at[wid])


def _sc_gather_max(y, z, gidx):
    mesh = plsc.VectorSubcoreMesh(core_axis_name="c", subcore_axis_name="s",
                                  num_cores=2, num_subcores=16)
    f = pl.kernel(
        _sc_gather_max_body,
        out_type=(
            jax.ShapeDtypeStruct((BNT, D), jnp.float32),
            jax.ShapeDtypeStruct((NW, D), jnp.float32),
            jax.ShapeDtypeStruct((NW, D), jnp.float32),
        ),
        mesh=mesh,
        scratch_types=[
            pltpu.VMEM((P * K,), jnp.int32),
            pltpu.VMEM_SHARED((BNT // 2, D), jnp.bfloat16),
            pltpu.VMEM((GK, D), jnp.bfloat16),
            pltpu.VMEM((GK, D), jnp.bfloat16),
            pltpu.VMEM((G, D), jnp.float32),
            pltpu.VMEM((G, D), jnp.float32),
            pltpu.VMEM((G, D), jnp.float32),
            pltpu.VMEM((G, D), jnp.float32),
            pltpu.VMEM((D,), jnp.float32),
            pltpu.VMEM((D,), jnp.float32),
            pltpu.SemaphoreType.DMA,
            pltpu.SemaphoreType.DMA,
            pltpu.SemaphoreType.DMA,
            pltpu.SemaphoreType.DMA,
            pltpu.SemaphoreType.DMA,
            pltpu.SemaphoreType.DMA,
        ],
        compiler_params=pltpu.CompilerParams(use_tc_tiling_on_sc=False,
                                             needs_layout_passes=False),
    )
    return f(y, z, gidx)


_LG = [b2 * 32 + (i // 2) + 16 * (i % 2) for b2 in range(2) for i in range(32)]


def kernel(x, neighbor_ind, W1, W2, gamma1, beta1, gamma2, beta2):
    lg = jnp.array(_LG, dtype=jnp.int32)
    w1y = W1[:, :D].T[:, lg]
    w1z = (W1[:, D:] - W1[:, :D]).T
    w2y = W2[:, :D].T[:, lg]
    w2z = (W2[:, D:] - W2[:, :D]).T
    gidx = (neighbor_ind.astype(jnp.int32)
            + ((jnp.arange(B, dtype=jnp.int32) % (B // 2)) * N)[:, None, None]
            ).reshape(BNT * K)
    g1 = gamma1.reshape(1, D)
    b1 = beta1.reshape(1, D)
    g2 = gamma2.reshape(1, D)
    b2 = beta2.reshape(1, D)

    y1, z1 = _mm_in(x, w1y, w1z)
    t1, ps1, pq1 = _sc_gather_max(y1, z1, gidx)
    y2, z2 = _mm_mid(t1, ps1, pq1, g1, b1, w2y, w2z, x)
    t2, ps2, pq2 = _sc_gather_max(y2, z2, gidx)
    return _mm_out(t2, ps2, pq2, g2, b2)


# speedup vs baseline: 1.0477x; 1.0477x over previous
import functools

import jax
import jax.numpy as jnp
from jax import lax
from jax.experimental import pallas as pl
from jax.experimental.pallas import tpu as pltpu
from jax.experimental.pallas import tpu_sc as plsc

B, D, N, K = 8, 64, 4096, 16
BNT = B * N
BT = 512
NB = N // BT
NW = 32
P = BNT // NW
G = 16
GK = G * K
NCH = P // G
NH = NCH // 2
L = 16
EPS = 1e-5


def _gelu(v):
    a1, a2, a3, a4, a5 = (0.254829592, -0.284496736, 1.421413741,
                          -1.453152027, 1.061405429)
    p = 0.3275911
    u = v * 0.7071067811865476
    s = jnp.sign(u)
    ua = jnp.abs(u)
    t = 1.0 / (1.0 + p * ua)
    poly = ((((a5 * t + a4) * t + a3) * t + a2) * t + a1) * t
    erf = s * (1.0 - poly * jnp.exp(-ua * ua))
    return 0.5 * v * (1.0 + erf)


def _mm_in_body(x_ref, wy_ref, wz_ref, y_ref, z_ref):
    xb = x_ref[0]
    dn = (((0,), (0,)), ((), ()))
    y_ref[...] = lax.dot_general(
        xb, wy_ref[...], dn,
        preferred_element_type=jnp.float32).astype(jnp.bfloat16)
    z_ref[...] = lax.dot_general(xb, wz_ref[...], dn,
                                 preferred_element_type=jnp.float32)


def _mm_in(x, wy, wz):
    return pl.pallas_call(
        _mm_in_body,
        grid=(B, NB),
        in_specs=[
            pl.BlockSpec((1, D, BT), lambda b, j: (b, 0, j)),
            pl.BlockSpec((D, D), lambda b, j: (0, 0)),
            pl.BlockSpec((D, D), lambda b, j: (0, 0)),
        ],
        out_specs=[pl.BlockSpec((BT, D), lambda b, j: (b * NB + j, 0))] * 2,
        out_shape=[jax.ShapeDtypeStruct((BNT, D), jnp.bfloat16),
                   jax.ShapeDtypeStruct((BNT, D), jnp.float32)],
    )(x, wy, wz)


def _bn_coeffs(ps, pq, g, bt):
    ssum = jnp.sum(ps, axis=0)
    ssq = jnp.sum(pq, axis=0)
    mean = ssum * (1.0 / BNT)
    var = ssq * (1.0 / BNT) - mean * mean
    scale = g[0] * lax.rsqrt(var + EPS)
    shift = bt[0] - mean * scale
    return scale, shift


def _mm_mid_body(t_ref, ps_ref, pq_ref, g_ref, b_ref, wy_ref, wz_ref, x_ref,
                 y_ref, z_ref):
    scale, shift = _bn_coeffs(ps_ref[...], pq_ref[...], g_ref[...], b_ref[...])
    h = _gelu(t_ref[...] * scale[None, :] + shift[None, :])
    dn = (((1,), (0,)), ((), ()))
    y_ref[...] = lax.dot_general(
        h, wy_ref[...], dn,
        preferred_element_type=jnp.float32).astype(jnp.bfloat16)
    z_ref[...] = lax.dot_general(h, wz_ref[...], dn,
                                 preferred_element_type=jnp.float32) \
        + jnp.transpose(x_ref[0])


def _mm_mid(t1, ps, pq, g, bt, wy, wz, x):
    return pl.pallas_call(
        _mm_mid_body,
        grid=(B, NB),
        in_specs=[
            pl.BlockSpec((BT, D), lambda b, j: (b * NB + j, 0)),
            pl.BlockSpec((NW, D), lambda b, j: (0, 0)),
            pl.BlockSpec((NW, D), lambda b, j: (0, 0)),
            pl.BlockSpec((1, D), lambda b, j: (0, 0)),
            pl.BlockSpec((1, D), lambda b, j: (0, 0)),
            pl.BlockSpec((D, D), lambda b, j: (0, 0)),
            pl.BlockSpec((D, D), lambda b, j: (0, 0)),
            pl.BlockSpec((1, D, BT), lambda b, j: (b, 0, j)),
        ],
        out_specs=[pl.BlockSpec((BT, D), lambda b, j: (b * NB + j, 0))] * 2,
        out_shape=[jax.ShapeDtypeStruct((BNT, D), jnp.bfloat16),
                   jax.ShapeDtypeStruct((BNT, D), jnp.float32)],
    )(t1, ps, pq, g, bt, wy, wz, x)


def _mm_out_body(t_ref, ps_ref, pq_ref, g_ref, b_ref, out_ref):
    scale, shift = _bn_coeffs(ps_ref[...], pq_ref[...], g_ref[...], b_ref[...])
    r = _gelu(t_ref[...] * scale[None, :] + shift[None, :])
    out_ref[0] = jnp.transpose(r)


def _mm_out(t2, ps, pq, g, bt):
    return pl.pallas_call(
        _mm_out_body,
        grid=(B, NB),
        in_specs=[
            pl.BlockSpec((BT, D), lambda b, j: (b * NB + j, 0)),
            pl.BlockSpec((NW, D), lambda b, j: (0, 0)),
            pl.BlockSpec((NW, D), lambda b, j: (0, 0)),
            pl.BlockSpec((1, D), lambda b, j: (0, 0)),
            pl.BlockSpec((1, D), lambda b, j: (0, 0)),
        ],
        out_specs=pl.BlockSpec((1, D, BT), lambda b, j: (b, 0, j)),
        out_shape=jax.ShapeDtypeStruct((B, D, N), jnp.float32),
    )(t2, ps, pq, g, bt)


def _sc_gather_max_body(y_hbm, z_hbm, gidx_hbm, t_hbm, pss_hbm, psq_hbm,
                        idx_v, ysh, rows0, rows1, z0, z1, t0, t1,
                        accs_v, accq_v, sg0, sg1, sz0, sz1, sw0, sw1):
    cid = lax.axis_index("c")
    sid = lax.axis_index("s")
    wid = cid * 16 + sid
    base = wid * P

    hb = BNT // 2
    st = hb // 16
    nst = st // GK
    for ss in range(nst):
        lo = sid * st + ss * GK
        pltpu.sync_copy(y_hbm.at[pl.ds(cid * hb + lo, GK)], rows0)
        pltpu.sync_copy(rows0, ysh.at[pl.ds(lo, GK)])
    pltpu.sync_copy(gidx_hbm.at[pl.ds(base * K, P * K)], idx_v)
    plsc.subcore_barrier()

    def g_copy(c, rows_v, sem):
        return pltpu.make_async_copy(
            ysh.at[idx_v.at[pl.ds(c * GK, GK)]], rows_v, sem)

    def z_copy(c, z_v, sem):
        return pltpu.make_async_copy(z_hbm.at[pl.ds(base + c * G, G)], z_v, sem)

    def w_copy(c, t_v, sem):
        return pltpu.make_async_copy(t_v, t_hbm.at[pl.ds(base + c * G, G)], sem)

    def compute(rows_v, z_v, t_v, accs):
        new = list(accs)
        for i in range(G):
            for j2 in range(D // (2 * L)):
                sl = pl.ds(2 * L * j2, 2 * L)
                m = rows_v[i * K, sl]
                for kk in range(1, K):
                    m = jnp.maximum(m, rows_v[i * K + kk, sl])
                ga, gb = plsc.unpack(m, format=plsc.PackFormat.INTERLEAVED)
                for j, gv in ((2 * j2, ga), (2 * j2 + 1, gb)):
                    sj = pl.ds(L * j, L)
                    t = gv + z_v[i, sj]
                    t_v[i, sj] = t
                    new[j] = new[j] + t
                    new[4 + j] = new[4 + j] + t * t
        return tuple(new)

    g_copy(0, rows0, sg0).start()
    z_copy(0, z0, sz0).start()

    zero = jnp.zeros((L,), jnp.float32)

    def body(s, accs):
        c0 = 2 * s
        c1 = c0 + 1
        g_copy(c1, rows1, sg1).start()
        z_copy(c1, z1, sz1).start()
        g_copy(c0, rows0, sg0).wait()
        z_copy(c0, z0, sz0).wait()

        @pl.when(s > 0)
        def _():
            w_copy(c0 - 2, t0, sw0).wait()

        accs = compute(rows0, z0, t0, accs)
        w_copy(c0, t0, sw0).start()

        @pl.when(s + 1 < NH)
        def _():
            g_copy(c0 + 2, rows0, sg0).start()
            z_copy(c0 + 2, z0, sz0).start()

        g_copy(c1, rows1, sg1).wait()
        z_copy(c1, z1, sz1).wait()

        @pl.when(s > 0)
        def _():
            w_copy(c1 - 2, t1, sw1).wait()

        accs = compute(rows1, z1, t1, accs)
        w_copy(c1, t1, sw1).start()
        return accs

    accs = lax.fori_loop(0, NH, body, tuple(zero for _ in range(8)))
    w_copy(NCH - 2, t0, sw0).wait()
    w_copy(NCH - 1, t1, sw1).wait()
    for j in range(D // L):
        accs_v[pl.ds(L * j, L)] = accs[j]
        accq_v[pl.ds(L * j, L)] = accs[4 + j]
    pltpu.sync_copy(accs_v, pss_hbm.at[wid])
    pltpu.sync_copy(accq_v, psq_hbm.at[wid])


def _sc_gather_max(y, z, gidx):
    mesh = plsc.VectorSubcoreMesh(core_axis_name="c", subcore_axis_name="s",
                                  num_cores=2, num_subcores=16)
    f = pl.kernel(
        _sc_gather_max_body,
        out_type=(
            jax.ShapeDtypeStruct((BNT, D), jnp.float32),
            jax.ShapeDtypeStruct((NW, D), jnp.float32),
            jax.ShapeDtypeStruct((NW, D), jnp.float32),
        ),
        mesh=mesh,
        scratch_types=[
            pltpu.VMEM((P * K,), jnp.int32),
            pltpu.VMEM_SHARED((BNT // 2, D), jnp.bfloat16),
            pltpu.VMEM((GK, D), jnp.bfloat16),
            pltpu.VMEM((GK, D), jnp.bfloat16),
            pltpu.VMEM((G, D), jnp.float32),
            pltpu.VMEM((G, D), jnp.float32),
            pltpu.VMEM((G, D), jnp.float32),
            pltpu.VMEM((G, D), jnp.float32),
            pltpu.VMEM((D,), jnp.float32),
            pltpu.VMEM((D,), jnp.float32),
            pltpu.SemaphoreType.DMA,
            pltpu.SemaphoreType.DMA,
            pltpu.SemaphoreType.DMA,
            pltpu.SemaphoreType.DMA,
            pltpu.SemaphoreType.DMA,
            pltpu.SemaphoreType.DMA,
        ],
        compiler_params=pltpu.CompilerParams(use_tc_tiling_on_sc=False,
                                             needs_layout_passes=False),
    )
    return f(y, z, gidx)


_LG = [b2 * 32 + (i // 2) + 16 * (i % 2) for b2 in range(2) for i in range(32)]


def kernel(x, neighbor_ind, W1, W2, gamma1, beta1, gamma2, beta2):
    lg = jnp.array(_LG, dtype=jnp.int32)
    w1y = W1[:, :D].T[:, lg]
    w1z = (W1[:, D:] - W1[:, :D]).T
    w2y = W2[:, :D].T[:, lg]
    w2z = (W2[:, D:] - W2[:, :D]).T
    gidx = (neighbor_ind.astype(jnp.int32)
            + ((jnp.arange(B, dtype=jnp.int32) % (B // 2)) * N)[:, None, None]
            ).reshape(BNT * K)
    g1 = gamma1.reshape(1, D)
    b1 = beta1.reshape(1, D)
    g2 = gamma2.reshape(1, D)
    b2 = beta2.reshape(1, D)

    y1, z1 = _mm_in(x, w1y, w1z)
    t1, ps1, pq1 = _sc_gather_max(y1, z1, gidx)
    y2, z2 = _mm_mid(t1, ps1, pq1, g1, b1, w2y, w2z, x)
    t2, ps2, pq2 = _sc_gather_max(y2, z2, gidx)
    return _mm_out(t2, ps2, pq2, g2, b2)


# speedup vs baseline: 1.1566x; 1.1039x over previous
import functools

import jax
import jax.numpy as jnp
from jax import lax
from jax.experimental import pallas as pl
from jax.experimental.pallas import tpu as pltpu
from jax.experimental.pallas import tpu_sc as plsc

B, D, N, K = 8, 64, 4096, 16
BNT = B * N
BT = 512
NB = N // BT
NW = 32
P = BNT // NW
G = 16
GK = G * K
NCH = P // G
NH = NCH // 2
L = 16
EPS = 1e-5


def _gelu(v):
    a1, a2, a3, a4, a5 = (0.254829592, -0.284496736, 1.421413741,
                          -1.453152027, 1.061405429)
    p = 0.3275911
    u = v * 0.7071067811865476
    s = jnp.sign(u)
    ua = jnp.abs(u)
    t = 1.0 / (1.0 + p * ua)
    poly = ((((a5 * t + a4) * t + a3) * t + a2) * t + a1) * t
    erf = s * (1.0 - poly * jnp.exp(-ua * ua))
    return 0.5 * v * (1.0 + erf)


def _mm_in_body(x_ref, wy_ref, wz_ref, y_ref, z_ref):
    xb = x_ref[0]
    dn = (((0,), (0,)), ((), ()))
    y_ref[...] = lax.dot_general(
        xb, wy_ref[...], dn,
        preferred_element_type=jnp.float32).astype(jnp.bfloat16)
    z_ref[...] = lax.dot_general(xb, wz_ref[...], dn,
                                 preferred_element_type=jnp.float32)


def _mm_in(x, wy, wz):
    return pl.pallas_call(
        _mm_in_body,
        grid=(B, NB),
        in_specs=[
            pl.BlockSpec((1, D, BT), lambda b, j: (b, 0, j)),
            pl.BlockSpec((D, D), lambda b, j: (0, 0)),
            pl.BlockSpec((D, D), lambda b, j: (0, 0)),
        ],
        out_specs=[pl.BlockSpec((BT, D), lambda b, j: (b * NB + j, 0))] * 2,
        out_shape=[jax.ShapeDtypeStruct((BNT, D), jnp.bfloat16),
                   jax.ShapeDtypeStruct((BNT, D), jnp.float32)],
    )(x, wy, wz)


def _bn_coeffs(ps, pq, g, bt):
    ssum = jnp.sum(ps, axis=0)
    ssq = jnp.sum(pq, axis=0)
    mean = ssum * (1.0 / BNT)
    var = ssq * (1.0 / BNT) - mean * mean
    scale = g[0] * lax.rsqrt(var + EPS)
    shift = bt[0] - mean * scale
    return scale, shift


def _mm_mid_body(t_ref, ps_ref, pq_ref, g_ref, b_ref, wy_ref, wz_ref, x_ref,
                 y_ref, z_ref):
    scale, shift = _bn_coeffs(ps_ref[...], pq_ref[...], g_ref[...], b_ref[...])
    h = _gelu(t_ref[...] * scale[None, :] + shift[None, :])
    dn = (((1,), (0,)), ((), ()))
    y_ref[...] = lax.dot_general(
        h, wy_ref[...], dn,
        preferred_element_type=jnp.float32).astype(jnp.bfloat16)
    z_ref[...] = lax.dot_general(h, wz_ref[...], dn,
                                 preferred_element_type=jnp.float32) \
        + jnp.transpose(x_ref[0])


def _mm_mid(t1, ps, pq, g, bt, wy, wz, x):
    return pl.pallas_call(
        _mm_mid_body,
        grid=(B, NB),
        in_specs=[
            pl.BlockSpec((BT, D), lambda b, j: (b * NB + j, 0)),
            pl.BlockSpec((NW, D), lambda b, j: (0, 0)),
            pl.BlockSpec((NW, D), lambda b, j: (0, 0)),
            pl.BlockSpec((1, D), lambda b, j: (0, 0)),
            pl.BlockSpec((1, D), lambda b, j: (0, 0)),
            pl.BlockSpec((D, D), lambda b, j: (0, 0)),
            pl.BlockSpec((D, D), lambda b, j: (0, 0)),
            pl.BlockSpec((1, D, BT), lambda b, j: (b, 0, j)),
        ],
        out_specs=[pl.BlockSpec((BT, D), lambda b, j: (b * NB + j, 0))] * 2,
        out_shape=[jax.ShapeDtypeStruct((BNT, D), jnp.bfloat16),
                   jax.ShapeDtypeStruct((BNT, D), jnp.float32)],
    )(t1, ps, pq, g, bt, wy, wz, x)


def _mm_out_body(t_ref, ps_ref, pq_ref, g_ref, b_ref, out_ref):
    scale, shift = _bn_coeffs(ps_ref[...], pq_ref[...], g_ref[...], b_ref[...])
    r = _gelu(t_ref[...] * scale[None, :] + shift[None, :])
    out_ref[0] = jnp.transpose(r)


def _mm_out(t2, ps, pq, g, bt):
    return pl.pallas_call(
        _mm_out_body,
        grid=(B, NB),
        in_specs=[
            pl.BlockSpec((BT, D), lambda b, j: (b * NB + j, 0)),
            pl.BlockSpec((NW, D), lambda b, j: (0, 0)),
            pl.BlockSpec((NW, D), lambda b, j: (0, 0)),
            pl.BlockSpec((1, D), lambda b, j: (0, 0)),
            pl.BlockSpec((1, D), lambda b, j: (0, 0)),
        ],
        out_specs=pl.BlockSpec((1, D, BT), lambda b, j: (b, 0, j)),
        out_shape=jax.ShapeDtypeStruct((B, D, N), jnp.float32),
    )(t2, ps, pq, g, bt)


def _sc_gather_max_body(y_hbm, z_hbm, gidx_hbm, t_hbm, pss_hbm, psq_hbm,
                        ysh, rows0, rows1, i0, i1, z0, z1, t0, t1,
                        accs_v, accq_v,
                        sg0, sg1, si0, si1, sz0, sz1, sw0, sw1):
    cid = lax.axis_index("c")
    sid = lax.axis_index("s")
    wid = cid * 16 + sid
    base = wid * P

    hb = BNT // 2
    st = hb // 16
    nst = st // GK
    for ss in range(nst):
        lo = sid * st + ss * GK
        pltpu.sync_copy(y_hbm.at[pl.ds(cid * hb + lo, GK)], rows0)
        pltpu.sync_copy(rows0, ysh.at[pl.ds(lo, GK)])
    plsc.subcore_barrier()

    def i_copy(c, i_v, sem):
        return pltpu.make_async_copy(
            gidx_hbm.at[pl.ds((base + c * G) * K, GK)], i_v, sem)

    def g_copy(i_v, rows_v, sem):
        return pltpu.make_async_copy(ysh.at[i_v], rows_v, sem)

    def z_copy(c, z_v, sem):
        return pltpu.make_async_copy(z_hbm.at[pl.ds(base + c * G, G)], z_v, sem)

    def w_copy(c, t_v, sem):
        return pltpu.make_async_copy(t_v, t_hbm.at[pl.ds(base + c * G, G)], sem)

    def compute(rows_v, z_v, t_v, accs):
        new = list(accs)
        for i in range(G):
            for j2 in range(D // (2 * L)):
                sl = pl.ds(2 * L * j2, 2 * L)
                m = rows_v[i * K, sl]
                for kk in range(1, K):
                    m = jnp.maximum(m, rows_v[i * K + kk, sl])
                ga, gb = plsc.unpack(m, format=plsc.PackFormat.INTERLEAVED)
                for j, gv in ((2 * j2, ga), (2 * j2 + 1, gb)):
                    sj = pl.ds(L * j, L)
                    t = gv + z_v[i, sj]
                    t_v[i, sj] = t
                    new[j] = new[j] + t
                    new[4 + j] = new[4 + j] + t * t
        return tuple(new)

    i_copy(0, i0, si0).start()
    i_copy(1, i1, si1).start()
    z_copy(0, z0, sz0).start()
    i_copy(0, i0, si0).wait()
    g_copy(i0, rows0, sg0).start()

    zero = jnp.zeros((L,), jnp.float32)

    def body(s, accs):
        c0 = 2 * s
        c1 = c0 + 1
        i_copy(c1, i1, si1).wait()
        g_copy(i1, rows1, sg1).start()
        z_copy(c1, z1, sz1).start()

        g_copy(i0, rows0, sg0).wait()

        @pl.when(s + 1 < NH)
        def _():
            i_copy(c0 + 2, i0, si0).start()

        z_copy(c0, z0, sz0).wait()

        @pl.when(s > 0)
        def _():
            w_copy(c0 - 2, t0, sw0).wait()

        accs = compute(rows0, z0, t0, accs)
        w_copy(c0, t0, sw0).start()

        g_copy(i1, rows1, sg1).wait()

        @pl.when(s + 1 < NH)
        def _():
            i_copy(c1 + 2, i1, si1).start()
            i_copy(c0 + 2, i0, si0).wait()
            g_copy(i0, rows0, sg0).start()
            z_copy(c0 + 2, z0, sz0).start()

        z_copy(c1, z1, sz1).wait()

        @pl.when(s > 0)
        def _():
            w_copy(c1 - 2, t1, sw1).wait()

        accs = compute(rows1, z1, t1, accs)
        w_copy(c1, t1, sw1).start()
        return accs

    accs = lax.fori_loop(0, NH, body, tuple(zero for _ in range(8)))
    w_copy(NCH - 2, t0, sw0).wait()
    w_copy(NCH - 1, t1, sw1).wait()
    for j in range(D // L):
        accs_v[pl.ds(L * j, L)] = accs[j]
        accq_v[pl.ds(L * j, L)] = accs[4 + j]
    pltpu.sync_copy(accs_v, pss_hbm.at[wid])
    pltpu.sync_copy(accq_v, psq_hbm.at[wid])


def _sc_gather_max(y, z, gidx):
    mesh = plsc.VectorSubcoreMesh(core_axis_name="c", subcore_axis_name="s",
                                  num_cores=2, num_subcores=16)
    f = pl.kernel(
        _sc_gather_max_body,
        out_type=(
            jax.ShapeDtypeStruct((BNT, D), jnp.float32),
            jax.ShapeDtypeStruct((NW, D), jnp.float32),
            jax.ShapeDtypeStruct((NW, D), jnp.float32),
        ),
        mesh=mesh,
        scratch_types=[
            pltpu.VMEM_SHARED((BNT // 2, D), jnp.bfloat16),
            pltpu.VMEM((GK, D), jnp.bfloat16),
            pltpu.VMEM((GK, D), jnp.bfloat16),
            pltpu.VMEM((GK,), jnp.int32),
            pltpu.VMEM((GK,), jnp.int32),
            pltpu.VMEM((G, D), jnp.float32),
            pltpu.VMEM((G, D), jnp.float32),
            pltpu.VMEM((G, D), jnp.float32),
            pltpu.VMEM((G, D), jnp.float32),
            pltpu.VMEM((D,), jnp.float32),
            pltpu.VMEM((D,), jnp.float32),
            pltpu.SemaphoreType.DMA,
            pltpu.SemaphoreType.DMA,
            pltpu.SemaphoreType.DMA,
            pltpu.SemaphoreType.DMA,
            pltpu.SemaphoreType.DMA,
            pltpu.SemaphoreType.DMA,
            pltpu.SemaphoreType.DMA,
            pltpu.SemaphoreType.DMA,
        ],
        compiler_params=pltpu.CompilerParams(use_tc_tiling_on_sc=False,
                                             needs_layout_passes=False),
    )
    return f(y, z, gidx)


_LG = [b2 * 32 + (i // 2) + 16 * (i % 2) for b2 in range(2) for i in range(32)]


def kernel(x, neighbor_ind, W1, W2, gamma1, beta1, gamma2, beta2):
    lg = jnp.array(_LG, dtype=jnp.int32)
    w1y = W1[:, :D].T[:, lg]
    w1z = (W1[:, D:] - W1[:, :D]).T
    w2y = W2[:, :D].T[:, lg]
    w2z = (W2[:, D:] - W2[:, :D]).T
    gidx = (neighbor_ind.astype(jnp.int32)
            + ((jnp.arange(B, dtype=jnp.int32) % (B // 2)) * N)[:, None, None]
            ).reshape(BNT * K)
    g1 = gamma1.reshape(1, D)
    b1 = beta1.reshape(1, D)
    g2 = gamma2.reshape(1, D)
    b2 = beta2.reshape(1, D)

    y1, z1 = _mm_in(x, w1y, w1z)
    t1, ps1, pq1 = _sc_gather_max(y1, z1, gidx)
    y2, z2 = _mm_mid(t1, ps1, pq1, g1, b1, w2y, w2z, x)
    t2, ps2, pq2 = _sc_gather_max(y2, z2, gidx)
    return _mm_out(t2, ps2, pq2, g2, b2)


# speedup vs baseline: 1.3458x; 1.1637x over previous
import functools

import jax
import jax.numpy as jnp
from jax import lax
from jax.experimental import pallas as pl
from jax.experimental.pallas import tpu as pltpu
from jax.experimental.pallas import tpu_sc as plsc

B, D, N, K = 8, 64, 4096, 16
BNT = B * N
BT = 2048
NB = N // BT
NW = 32
P = BNT // NW
G = 16
GK = G * K
NCH = P // G
NH = NCH // 2
L = 16
EPS = 1e-5


def _gelu(v):
    a1, a2, a3, a4, a5 = (0.254829592, -0.284496736, 1.421413741,
                          -1.453152027, 1.061405429)
    p = 0.3275911
    u = v * 0.7071067811865476
    s = jnp.sign(u)
    ua = jnp.abs(u)
    t = 1.0 / (1.0 + p * ua)
    poly = ((((a5 * t + a4) * t + a3) * t + a2) * t + a1) * t
    erf = s * (1.0 - poly * jnp.exp(-ua * ua))
    return 0.5 * v * (1.0 + erf)


def _mm_in_body(x_ref, wy_ref, wz_ref, y_ref, z_ref):
    xb = x_ref[0]
    dn = (((0,), (0,)), ((), ()))
    y_ref[...] = lax.dot_general(
        xb, wy_ref[...], dn,
        preferred_element_type=jnp.float32).astype(jnp.bfloat16)
    z_ref[...] = lax.dot_general(xb, wz_ref[...], dn,
                                 preferred_element_type=jnp.float32)


def _mm_in(x, wy, wz):
    return pl.pallas_call(
        _mm_in_body,
        grid=(B, NB),
        in_specs=[
            pl.BlockSpec((1, D, BT), lambda b, j: (b, 0, j)),
            pl.BlockSpec((D, D), lambda b, j: (0, 0)),
            pl.BlockSpec((D, D), lambda b, j: (0, 0)),
        ],
        out_specs=[pl.BlockSpec((BT, D), lambda b, j: (b * NB + j, 0))] * 2,
        out_shape=[jax.ShapeDtypeStruct((BNT, D), jnp.bfloat16),
                   jax.ShapeDtypeStruct((BNT, D), jnp.float32)],
    )(x, wy, wz)


def _bn_coeffs(ps, pq, g, bt):
    ssum = jnp.sum(ps, axis=0)
    ssq = jnp.sum(pq, axis=0)
    mean = ssum * (1.0 / BNT)
    var = ssq * (1.0 / BNT) - mean * mean
    scale = g[0] * lax.rsqrt(var + EPS)
    shift = bt[0] - mean * scale
    return scale, shift


def _mm_mid_body(t_ref, ps_ref, pq_ref, g_ref, b_ref, wy_ref, wz_ref, x_ref,
                 y_ref, z_ref):
    scale, shift = _bn_coeffs(ps_ref[...], pq_ref[...], g_ref[...], b_ref[...])
    h = _gelu(t_ref[...] * scale[None, :] + shift[None, :])
    dn = (((1,), (0,)), ((), ()))
    y_ref[...] = lax.dot_general(
        h, wy_ref[...], dn,
        preferred_element_type=jnp.float32).astype(jnp.bfloat16)
    z_ref[...] = lax.dot_general(h, wz_ref[...], dn,
                                 preferred_element_type=jnp.float32) \
        + jnp.transpose(x_ref[0])


def _mm_mid(t1, ps, pq, g, bt, wy, wz, x):
    return pl.pallas_call(
        _mm_mid_body,
        grid=(B, NB),
        in_specs=[
            pl.BlockSpec((BT, D), lambda b, j: (b * NB + j, 0)),
            pl.BlockSpec((NW, D), lambda b, j: (0, 0)),
            pl.BlockSpec((NW, D), lambda b, j: (0, 0)),
            pl.BlockSpec((1, D), lambda b, j: (0, 0)),
            pl.BlockSpec((1, D), lambda b, j: (0, 0)),
            pl.BlockSpec((D, D), lambda b, j: (0, 0)),
            pl.BlockSpec((D, D), lambda b, j: (0, 0)),
            pl.BlockSpec((1, D, BT), lambda b, j: (b, 0, j)),
        ],
        out_specs=[pl.BlockSpec((BT, D), lambda b, j: (b * NB + j, 0))] * 2,
        out_shape=[jax.ShapeDtypeStruct((BNT, D), jnp.bfloat16),
                   jax.ShapeDtypeStruct((BNT, D), jnp.float32)],
    )(t1, ps, pq, g, bt, wy, wz, x)


def _mm_out_body(t_ref, ps_ref, pq_ref, g_ref, b_ref, out_ref):
    scale, shift = _bn_coeffs(ps_ref[...], pq_ref[...], g_ref[...], b_ref[...])
    r = _gelu(t_ref[...] * scale[None, :] + shift[None, :])
    out_ref[0] = jnp.transpose(r)


def _mm_out(t2, ps, pq, g, bt):
    return pl.pallas_call(
        _mm_out_body,
        grid=(B, NB),
        in_specs=[
            pl.BlockSpec((BT, D), lambda b, j: (b * NB + j, 0)),
            pl.BlockSpec((NW, D), lambda b, j: (0, 0)),
            pl.BlockSpec((NW, D), lambda b, j: (0, 0)),
            pl.BlockSpec((1, D), lambda b, j: (0, 0)),
            pl.BlockSpec((1, D), lambda b, j: (0, 0)),
        ],
        out_specs=pl.BlockSpec((1, D, BT), lambda b, j: (b, 0, j)),
        out_shape=jax.ShapeDtypeStruct((B, D, N), jnp.float32),
    )(t2, ps, pq, g, bt)


def _sc_gather_max_body(y_hbm, z_hbm, gidx_hbm, t_hbm, pss_hbm, psq_hbm,
                        ysh, rows0, rows1, i0, i1, z0, z1, t0, t1,
                        accs_v, accq_v,
                        sg0, sg1, si0, si1, sz0, sz1, sw0, sw1):
    cid = lax.axis_index("c")
    sid = lax.axis_index("s")
    wid = cid * 16 + sid
    base = wid * P

    hb = BNT // 2
    st = hb // 16
    nst = st // GK
    for ss in range(nst):
        lo = sid * st + ss * GK
        pltpu.sync_copy(y_hbm.at[pl.ds(cid * hb + lo, GK)], rows0)
        pltpu.sync_copy(rows0, ysh.at[pl.ds(lo, GK)])
    plsc.subcore_barrier()

    def i_copy(c, i_v, sem):
        return pltpu.make_async_copy(
            gidx_hbm.at[pl.ds((base + c * G) * K, GK)], i_v, sem)

    def g_copy(i_v, rows_v, sem):
        return pltpu.make_async_copy(ysh.at[i_v], rows_v, sem)

    def z_copy(c, z_v, sem):
        return pltpu.make_async_copy(z_hbm.at[pl.ds(base + c * G, G)], z_v, sem)

    def w_copy(c, t_v, sem):
        return pltpu.make_async_copy(t_v, t_hbm.at[pl.ds(base + c * G, G)], sem)

    def compute(rows_v, z_v, t_v, accs):
        new = list(accs)
        for i in range(G):
            for j2 in range(D // (2 * L)):
                sl = pl.ds(2 * L * j2, 2 * L)
                m = rows_v[i * K, sl]
                for kk in range(1, K):
                    m = jnp.maximum(m, rows_v[i * K + kk, sl])
                ga, gb = plsc.unpack(m, format=plsc.PackFormat.INTERLEAVED)
                for j, gv in ((2 * j2, ga), (2 * j2 + 1, gb)):
                    sj = pl.ds(L * j, L)
                    t = gv + z_v[i, sj]
                    t_v[i, sj] = t
                    new[j] = new[j] + t
                    new[4 + j] = new[4 + j] + t * t
        return tuple(new)

    i_copy(0, i0, si0).start()
    i_copy(1, i1, si1).start()
    z_copy(0, z0, sz0).start()
    i_copy(0, i0, si0).wait()
    g_copy(i0, rows0, sg0).start()

    zero = jnp.zeros((L,), jnp.float32)

    def body(s, accs):
        c0 = 2 * s
        c1 = c0 + 1
        i_copy(c1, i1, si1).wait()
        g_copy(i1, rows1, sg1).start()
        z_copy(c1, z1, sz1).start()

        g_copy(i0, rows0, sg0).wait()

        @pl.when(s + 1 < NH)
        def _():
            i_copy(c0 + 2, i0, si0).start()

        z_copy(c0, z0, sz0).wait()

        @pl.when(s > 0)
        def _():
            w_copy(c0 - 2, t0, sw0).wait()

        accs = compute(rows0, z0, t0, accs)
        w_copy(c0, t0, sw0).start()

        g_copy(i1, rows1, sg1).wait()

        @pl.when(s + 1 < NH)
        def _():
            i_copy(c1 + 2, i1, si1).start()
            i_copy(c0 + 2, i0, si0).wait()
            g_copy(i0, rows0, sg0).start()
            z_copy(c0 + 2, z0, sz0).start()

        z_copy(c1, z1, sz1).wait()

        @pl.when(s > 0)
        def _():
            w_copy(c1 - 2, t1, sw1).wait()

        accs = compute(rows1, z1, t1, accs)
        w_copy(c1, t1, sw1).start()
        return accs

    accs = lax.fori_loop(0, NH, body, tuple(zero for _ in range(8)))
    w_copy(NCH - 2, t0, sw0).wait()
    w_copy(NCH - 1, t1, sw1).wait()
    for j in range(D // L):
        accs_v[pl.ds(L * j, L)] = accs[j]
        accq_v[pl.ds(L * j, L)] = accs[4 + j]
    pltpu.sync_copy(accs_v, pss_hbm.at[wid])
    pltpu.sync_copy(accq_v, psq_hbm.at[wid])


def _sc_gather_max(y, z, gidx):
    mesh = plsc.VectorSubcoreMesh(core_axis_name="c", subcore_axis_name="s",
                                  num_cores=2, num_subcores=16)
    f = pl.kernel(
        _sc_gather_max_body,
        out_type=(
            jax.ShapeDtypeStruct((BNT, D), jnp.float32),
            jax.ShapeDtypeStruct((NW, D), jnp.float32),
            jax.ShapeDtypeStruct((NW, D), jnp.float32),
        ),
        mesh=mesh,
        scratch_types=[
            pltpu.VMEM_SHARED((BNT // 2, D), jnp.bfloat16),
            pltpu.VMEM((GK, D), jnp.bfloat16),
            pltpu.VMEM((GK, D), jnp.bfloat16),
            pltpu.VMEM((GK,), jnp.int32),
            pltpu.VMEM((GK,), jnp.int32),
            pltpu.VMEM((G, D), jnp.float32),
            pltpu.VMEM((G, D), jnp.float32),
            pltpu.VMEM((G, D), jnp.float32),
            pltpu.VMEM((G, D), jnp.float32),
            pltpu.VMEM((D,), jnp.float32),
            pltpu.VMEM((D,), jnp.float32),
            pltpu.SemaphoreType.DMA,
            pltpu.SemaphoreType.DMA,
            pltpu.SemaphoreType.DMA,
            pltpu.SemaphoreType.DMA,
            pltpu.SemaphoreType.DMA,
            pltpu.SemaphoreType.DMA,
            pltpu.SemaphoreType.DMA,
            pltpu.SemaphoreType.DMA,
        ],
        compiler_params=pltpu.CompilerParams(use_tc_tiling_on_sc=False,
                                             needs_layout_passes=False),
    )
    return f(y, z, gidx)


_LG = [b2 * 32 + (i // 2) + 16 * (i % 2) for b2 in range(2) for i in range(32)]


def kernel(x, neighbor_ind, W1, W2, gamma1, beta1, gamma2, beta2):
    lg = jnp.array(_LG, dtype=jnp.int32)
    w1y = W1[:, :D].T[:, lg]
    w1z = (W1[:, D:] - W1[:, :D]).T
    w2y = W2[:, :D].T[:, lg]
    w2z = (W2[:, D:] - W2[:, :D]).T
    gidx = (neighbor_ind.astype(jnp.int32)
            + ((jnp.arange(B, dtype=jnp.int32) % (B // 2)) * N)[:, None, None]
            ).reshape(BNT * K)
    g1 = gamma1.reshape(1, D)
    b1 = beta1.reshape(1, D)
    g2 = gamma2.reshape(1, D)
    b2 = beta2.reshape(1, D)

    y1, z1 = _mm_in(x, w1y, w1z)
    t1, ps1, pq1 = _sc_gather_max(y1, z1, gidx)
    y2, z2 = _mm_mid(t1, ps1, pq1, g1, b1, w2y, w2z, x)
    t2, ps2, pq2 = _sc_gather_max(y2, z2, gidx)
    return _mm_out(t2, ps2, pq2, g2, b2)


# speedup vs baseline: 1.3838x; 1.0282x over previous
import functools

import jax
import jax.numpy as jnp
from jax import lax
from jax.experimental import pallas as pl
from jax.experimental.pallas import tpu as pltpu
from jax.experimental.pallas import tpu_sc as plsc

B, D, N, K = 8, 64, 4096, 16
BNT = B * N
BT = 4096
NB = N // BT
NW = 32
P = BNT // NW
G = 16
GK = G * K
NCH = P // G
NH = NCH // 2
L = 16
EPS = 1e-5


def _gelu(v):
    a1, a2, a3, a4, a5 = (0.254829592, -0.284496736, 1.421413741,
                          -1.453152027, 1.061405429)
    p = 0.3275911
    u = v * 0.7071067811865476
    s = jnp.sign(u)
    ua = jnp.abs(u)
    t = 1.0 / (1.0 + p * ua)
    poly = ((((a5 * t + a4) * t + a3) * t + a2) * t + a1) * t
    erf = s * (1.0 - poly * jnp.exp(-ua * ua))
    return 0.5 * v * (1.0 + erf)


def _mm_in_body(x_ref, wy_ref, wz_ref, y_ref, z_ref):
    xb = x_ref[0]
    dn = (((0,), (0,)), ((), ()))
    y_ref[...] = lax.dot_general(
        xb, wy_ref[...], dn,
        preferred_element_type=jnp.float32).astype(jnp.bfloat16)
    z_ref[...] = lax.dot_general(xb, wz_ref[...], dn,
                                 preferred_element_type=jnp.float32)


def _mm_in(x, wy, wz):
    return pl.pallas_call(
        _mm_in_body,
        grid=(B, NB),
        in_specs=[
            pl.BlockSpec((1, D, BT), lambda b, j: (b, 0, j)),
            pl.BlockSpec((D, D), lambda b, j: (0, 0)),
            pl.BlockSpec((D, D), lambda b, j: (0, 0)),
        ],
        out_specs=[pl.BlockSpec((BT, D), lambda b, j: (b * NB + j, 0))] * 2,
        out_shape=[jax.ShapeDtypeStruct((BNT, D), jnp.bfloat16),
                   jax.ShapeDtypeStruct((BNT, D), jnp.float32)],
    )(x, wy, wz)


def _bn_coeffs(ps, pq, g, bt):
    ssum = jnp.sum(ps, axis=0)
    ssq = jnp.sum(pq, axis=0)
    mean = ssum * (1.0 / BNT)
    var = ssq * (1.0 / BNT) - mean * mean
    scale = g[0] * lax.rsqrt(var + EPS)
    shift = bt[0] - mean * scale
    return scale, shift


def _mm_mid_body(t_ref, ps_ref, pq_ref, g_ref, b_ref, wy_ref, wz_ref, x_ref,
                 y_ref, z_ref):
    scale, shift = _bn_coeffs(ps_ref[...], pq_ref[...], g_ref[...], b_ref[...])
    h = _gelu(t_ref[...] * scale[None, :] + shift[None, :])
    dn = (((1,), (0,)), ((), ()))
    y_ref[...] = lax.dot_general(
        h, wy_ref[...], dn,
        preferred_element_type=jnp.float32).astype(jnp.bfloat16)
    z_ref[...] = lax.dot_general(h, wz_ref[...], dn,
                                 preferred_element_type=jnp.float32) \
        + jnp.transpose(x_ref[0])


def _mm_mid(t1, ps, pq, g, bt, wy, wz, x):
    return pl.pallas_call(
        _mm_mid_body,
        grid=(B, NB),
        in_specs=[
            pl.BlockSpec((BT, D), lambda b, j: (b * NB + j, 0)),
            pl.BlockSpec((NW, D), lambda b, j: (0, 0)),
            pl.BlockSpec((NW, D), lambda b, j: (0, 0)),
            pl.BlockSpec((1, D), lambda b, j: (0, 0)),
            pl.BlockSpec((1, D), lambda b, j: (0, 0)),
            pl.BlockSpec((D, D), lambda b, j: (0, 0)),
            pl.BlockSpec((D, D), lambda b, j: (0, 0)),
            pl.BlockSpec((1, D, BT), lambda b, j: (b, 0, j)),
        ],
        out_specs=[pl.BlockSpec((BT, D), lambda b, j: (b * NB + j, 0))] * 2,
        out_shape=[jax.ShapeDtypeStruct((BNT, D), jnp.bfloat16),
                   jax.ShapeDtypeStruct((BNT, D), jnp.float32)],
    )(t1, ps, pq, g, bt, wy, wz, x)


def _mm_out_body(t_ref, ps_ref, pq_ref, g_ref, b_ref, out_ref):
    scale, shift = _bn_coeffs(ps_ref[...], pq_ref[...], g_ref[...], b_ref[...])
    r = _gelu(t_ref[...] * scale[None, :] + shift[None, :])
    out_ref[0] = jnp.transpose(r)


def _mm_out(t2, ps, pq, g, bt):
    return pl.pallas_call(
        _mm_out_body,
        grid=(B, NB),
        in_specs=[
            pl.BlockSpec((BT, D), lambda b, j: (b * NB + j, 0)),
            pl.BlockSpec((NW, D), lambda b, j: (0, 0)),
            pl.BlockSpec((NW, D), lambda b, j: (0, 0)),
            pl.BlockSpec((1, D), lambda b, j: (0, 0)),
            pl.BlockSpec((1, D), lambda b, j: (0, 0)),
        ],
        out_specs=pl.BlockSpec((1, D, BT), lambda b, j: (b, 0, j)),
        out_shape=jax.ShapeDtypeStruct((B, D, N), jnp.float32),
    )(t2, ps, pq, g, bt)


def _sc_gather_max_body(y_hbm, z_hbm, gidx_hbm, t_hbm, pss_hbm, psq_hbm,
                        ysh, rows0, rows1, i0, i1, z0, z1, t0, t1,
                        accs_v, accq_v,
                        sg0, sg1, si0, si1, sz0, sz1, sw0, sw1):
    cid = lax.axis_index("c")
    sid = lax.axis_index("s")
    wid = cid * 16 + sid
    base = wid * P

    hb = BNT // 2
    st = hb // 16
    nst = st // GK
    for ss in range(nst):
        lo = sid * st + ss * GK
        pltpu.sync_copy(y_hbm.at[pl.ds(cid * hb + lo, GK)], rows0)
        pltpu.sync_copy(rows0, ysh.at[pl.ds(lo, GK)])
    plsc.subcore_barrier()

    def i_copy(c, i_v, sem):
        return pltpu.make_async_copy(
            gidx_hbm.at[pl.ds((base + c * G) * K, GK)], i_v, sem)

    def g_copy(i_v, rows_v, sem):
        return pltpu.make_async_copy(ysh.at[i_v], rows_v, sem)

    def z_copy(c, z_v, sem):
        return pltpu.make_async_copy(z_hbm.at[pl.ds(base + c * G, G)], z_v, sem)

    def w_copy(c, t_v, sem):
        return pltpu.make_async_copy(t_v, t_hbm.at[pl.ds(base + c * G, G)], sem)

    def compute(rows_v, z_v, t_v, accs):
        new = list(accs)
        for i in range(G):
            for j2 in range(D // (2 * L)):
                sl = pl.ds(2 * L * j2, 2 * L)
                m = rows_v[i * K, sl]
                for kk in range(1, K):
                    m = jnp.maximum(m, rows_v[i * K + kk, sl])
                ga, gb = plsc.unpack(m, format=plsc.PackFormat.INTERLEAVED)
                for j, gv in ((2 * j2, ga), (2 * j2 + 1, gb)):
                    sj = pl.ds(L * j, L)
                    t = gv + z_v[i, sj]
                    t_v[i, sj] = t
                    new[j] = new[j] + t
                    new[4 + j] = new[4 + j] + t * t
        return tuple(new)

    i_copy(0, i0, si0).start()
    i_copy(1, i1, si1).start()
    z_copy(0, z0, sz0).start()
    i_copy(0, i0, si0).wait()
    g_copy(i0, rows0, sg0).start()

    zero = jnp.zeros((L,), jnp.float32)

    def body(s, accs):
        c0 = 2 * s
        c1 = c0 + 1
        i_copy(c1, i1, si1).wait()
        g_copy(i1, rows1, sg1).start()
        z_copy(c1, z1, sz1).start()

        g_copy(i0, rows0, sg0).wait()

        @pl.when(s + 1 < NH)
        def _():
            i_copy(c0 + 2, i0, si0).start()

        z_copy(c0, z0, sz0).wait()

        @pl.when(s > 0)
        def _():
            w_copy(c0 - 2, t0, sw0).wait()

        accs = compute(rows0, z0, t0, accs)
        w_copy(c0, t0, sw0).start()

        g_copy(i1, rows1, sg1).wait()

        @pl.when(s + 1 < NH)
        def _():
            i_copy(c1 + 2, i1, si1).start()
            i_copy(c0 + 2, i0, si0).wait()
            g_copy(i0, rows0, sg0).start()
            z_copy(c0 + 2, z0, sz0).start()

        z_copy(c1, z1, sz1).wait()

        @pl.when(s > 0)
        def _():
            w_copy(c1 - 2, t1, sw1).wait()

        accs = compute(rows1, z1, t1, accs)
        w_copy(c1, t1, sw1).start()
        return accs

    accs = lax.fori_loop(0, NH, body, tuple(zero for _ in range(8)))
    w_copy(NCH - 2, t0, sw0).wait()
    w_copy(NCH - 1, t1, sw1).wait()
    for j in range(D // L):
        accs_v[pl.ds(L * j, L)] = accs[j]
        accq_v[pl.ds(L * j, L)] = accs[4 + j]
    pltpu.sync_copy(accs_v, pss_hbm.at[wid])
    pltpu.sync_copy(accq_v, psq_hbm.at[wid])


def _sc_gather_max(y, z, gidx):
    mesh = plsc.VectorSubcoreMesh(core_axis_name="c", subcore_axis_name="s",
                                  num_cores=2, num_subcores=16)
    f = pl.kernel(
        _sc_gather_max_body,
        out_type=(
            jax.ShapeDtypeStruct((BNT, D), jnp.float32),
            jax.ShapeDtypeStruct((NW, D), jnp.float32),
            jax.ShapeDtypeStruct((NW, D), jnp.float32),
        ),
        mesh=mesh,
        scratch_types=[
            pltpu.VMEM_SHARED((BNT // 2, D), jnp.bfloat16),
            pltpu.VMEM((GK, D), jnp.bfloat16),
            pltpu.VMEM((GK, D), jnp.bfloat16),
            pltpu.VMEM((GK,), jnp.int32),
            pltpu.VMEM((GK,), jnp.int32),
            pltpu.VMEM((G, D), jnp.float32),
            pltpu.VMEM((G, D), jnp.float32),
            pltpu.VMEM((G, D), jnp.float32),
            pltpu.VMEM((G, D), jnp.float32),
            pltpu.VMEM((D,), jnp.float32),
            pltpu.VMEM((D,), jnp.float32),
            pltpu.SemaphoreType.DMA,
            pltpu.SemaphoreType.DMA,
            pltpu.SemaphoreType.DMA,
            pltpu.SemaphoreType.DMA,
            pltpu.SemaphoreType.DMA,
            pltpu.SemaphoreType.DMA,
            pltpu.SemaphoreType.DMA,
            pltpu.SemaphoreType.DMA,
        ],
        compiler_params=pltpu.CompilerParams(use_tc_tiling_on_sc=False,
                                             needs_layout_passes=False),
    )
    return f(y, z, gidx)


_LG = [b2 * 32 + (i // 2) + 16 * (i % 2) for b2 in range(2) for i in range(32)]


def kernel(x, neighbor_ind, W1, W2, gamma1, beta1, gamma2, beta2):
    lg = jnp.array(_LG, dtype=jnp.int32)
    w1y = W1[:, :D].T[:, lg]
    w1z = (W1[:, D:] - W1[:, :D]).T
    w2y = W2[:, :D].T[:, lg]
    w2z = (W2[:, D:] - W2[:, :D]).T
    gidx = (neighbor_ind.astype(jnp.int32)
            + ((jnp.arange(B, dtype=jnp.int32) % (B // 2)) * N)[:, None, None]
            ).reshape(BNT * K)
    g1 = gamma1.reshape(1, D)
    b1 = beta1.reshape(1, D)
    g2 = gamma2.reshape(1, D)
    b2 = beta2.reshape(1, D)

    y1, z1 = _mm_in(x, w1y, w1z)
    t1, ps1, pq1 = _sc_gather_max(y1, z1, gidx)
    y2, z2 = _mm_mid(t1, ps1, pq1, g1, b1, w2y, w2z, x)
    t2, ps2, pq2 = _sc_gather_max(y2, z2, gidx)
    return _mm_out(t2, ps2, pq2, g2, b2)


# speedup vs baseline: 1.4269x; 1.0311x over previous
import functools

import jax
import jax.numpy as jnp
from jax import lax
from jax.experimental import pallas as pl
from jax.experimental.pallas import tpu as pltpu
from jax.experimental.pallas import tpu_sc as plsc

B, D, N, K = 8, 64, 4096, 16
BNT = B * N
BT = 4096
NB = N // BT
NW = 32
P = BNT // NW
G = 16
GK = G * K
NCH = P // G
NH = NCH // 2
L = 16
EPS = 1e-5


def _gelu(v):
    a1, a2, a3, a4, a5 = (0.254829592, -0.284496736, 1.421413741,
                          -1.453152027, 1.061405429)
    p = 0.3275911
    u = v * 0.7071067811865476
    s = jnp.sign(u)
    ua = jnp.abs(u)
    t = 1.0 / (1.0 + p * ua)
    poly = ((((a5 * t + a4) * t + a3) * t + a2) * t + a1) * t
    erf = s * (1.0 - poly * jnp.exp(-ua * ua))
    return 0.5 * v * (1.0 + erf)


def _mm_in_body(x_ref, wy_ref, wz_ref, y_ref, z_ref):
    xb = x_ref[0]
    dn = (((0,), (0,)), ((), ()))
    y_ref[...] = lax.dot_general(
        xb, wy_ref[...], dn,
        preferred_element_type=jnp.float32).astype(jnp.bfloat16)
    z_ref[...] = lax.dot_general(xb, wz_ref[...], dn,
                                 preferred_element_type=jnp.float32)


def _mm_in(x, wy, wz):
    return pl.pallas_call(
        _mm_in_body,
        grid=(B, NB),
        in_specs=[
            pl.BlockSpec((1, D, BT), lambda b, j: (b, 0, j)),
            pl.BlockSpec((D, D), lambda b, j: (0, 0)),
            pl.BlockSpec((D, D), lambda b, j: (0, 0)),
        ],
        out_specs=[pl.BlockSpec((BT, D), lambda b, j: (b * NB + j, 0))] * 2,
        out_shape=[jax.ShapeDtypeStruct((BNT, D), jnp.bfloat16),
                   jax.ShapeDtypeStruct((BNT, D), jnp.float32)],
    )(x, wy, wz)


def _bn_coeffs(ps, pq, g, bt):
    ssum = jnp.sum(ps, axis=0)
    ssq = jnp.sum(pq, axis=0)
    mean = ssum * (1.0 / BNT)
    var = ssq * (1.0 / BNT) - mean * mean
    scale = g[0] * lax.rsqrt(var + EPS)
    shift = bt[0] - mean * scale
    return scale, shift


def _mm_mid_body(t_ref, ps_ref, pq_ref, g_ref, b_ref, wy_ref, wz_ref, x_ref,
                 y_ref, z_ref):
    scale, shift = _bn_coeffs(ps_ref[...], pq_ref[...], g_ref[...], b_ref[...])
    h = _gelu(t_ref[...] * scale[None, :] + shift[None, :])
    dn = (((1,), (0,)), ((), ()))
    y_ref[...] = lax.dot_general(
        h, wy_ref[...], dn,
        preferred_element_type=jnp.float32).astype(jnp.bfloat16)
    z_ref[...] = lax.dot_general(h, wz_ref[...], dn,
                                 preferred_element_type=jnp.float32) \
        + jnp.transpose(x_ref[0])


def _mm_mid(t1, ps, pq, g, bt, wy, wz, x):
    return pl.pallas_call(
        _mm_mid_body,
        grid=(B, NB),
        in_specs=[
            pl.BlockSpec((BT, D), lambda b, j: (b * NB + j, 0)),
            pl.BlockSpec((NW, D), lambda b, j: (0, 0)),
            pl.BlockSpec((NW, D), lambda b, j: (0, 0)),
            pl.BlockSpec((1, D), lambda b, j: (0, 0)),
            pl.BlockSpec((1, D), lambda b, j: (0, 0)),
            pl.BlockSpec((D, D), lambda b, j: (0, 0)),
            pl.BlockSpec((D, D), lambda b, j: (0, 0)),
            pl.BlockSpec((1, D, BT), lambda b, j: (b, 0, j)),
        ],
        out_specs=[pl.BlockSpec((BT, D), lambda b, j: (b * NB + j, 0))] * 2,
        out_shape=[jax.ShapeDtypeStruct((BNT, D), jnp.bfloat16),
                   jax.ShapeDtypeStruct((BNT, D), jnp.float32)],
    )(t1, ps, pq, g, bt, wy, wz, x)


def _mm_out_body(t_ref, ps_ref, pq_ref, g_ref, b_ref, out_ref):
    scale, shift = _bn_coeffs(ps_ref[...], pq_ref[...], g_ref[...], b_ref[...])
    r = _gelu(t_ref[...] * scale[None, :] + shift[None, :])
    out_ref[0] = jnp.transpose(r)


def _mm_out(t2, ps, pq, g, bt):
    return pl.pallas_call(
        _mm_out_body,
        grid=(B, NB),
        in_specs=[
            pl.BlockSpec((BT, D), lambda b, j: (b * NB + j, 0)),
            pl.BlockSpec((NW, D), lambda b, j: (0, 0)),
            pl.BlockSpec((NW, D), lambda b, j: (0, 0)),
            pl.BlockSpec((1, D), lambda b, j: (0, 0)),
            pl.BlockSpec((1, D), lambda b, j: (0, 0)),
        ],
        out_specs=pl.BlockSpec((1, D, BT), lambda b, j: (b, 0, j)),
        out_shape=jax.ShapeDtypeStruct((B, D, N), jnp.float32),
    )(t2, ps, pq, g, bt)


def _sc_gather_max_body(y_hbm, z_hbm, gidx_hbm, t_hbm, pss_hbm, psq_hbm,
                        ysh, rows0, rows1, i0, i1, z0, z1, t0, t1,
                        accs_v, accq_v,
                        sg0, sg1, si0, si1, sz0, sz1, sw0, sw1):
    cid = lax.axis_index("c")
    sid = lax.axis_index("s")
    wid = cid * 16 + sid
    base = wid * P

    def i_copy(c, i_v, sem):
        return pltpu.make_async_copy(
            gidx_hbm.at[pl.ds((base + c * G) * K, GK)], i_v, sem)

    i_copy(0, i0, si0).start()
    i_copy(1, i1, si1).start()

    hb = BNT // 2
    st = hb // 16
    pltpu.sync_copy(y_hbm.at[pl.ds(cid * hb + sid * st, st)],
                    ysh.at[pl.ds(sid * st, st)])
    plsc.subcore_barrier()

    def g_copy(i_v, rows_v, sem):
        return pltpu.make_async_copy(ysh.at[i_v], rows_v, sem)

    def z_copy(c, z_v, sem):
        return pltpu.make_async_copy(z_hbm.at[pl.ds(base + c * G, G)], z_v, sem)

    def w_copy(c, t_v, sem):
        return pltpu.make_async_copy(t_v, t_hbm.at[pl.ds(base + c * G, G)], sem)

    def compute(rows_v, z_v, t_v, accs):
        new = list(accs)
        for i in range(G):
            for j2 in range(D // (2 * L)):
                sl = pl.ds(2 * L * j2, 2 * L)
                m = rows_v[i * K, sl]
                for kk in range(1, K):
                    m = jnp.maximum(m, rows_v[i * K + kk, sl])
                ga, gb = plsc.unpack(m, format=plsc.PackFormat.INTERLEAVED)
                for j, gv in ((2 * j2, ga), (2 * j2 + 1, gb)):
                    sj = pl.ds(L * j, L)
                    t = gv + z_v[i, sj]
                    t_v[i, sj] = t
                    new[j] = new[j] + t
                    new[4 + j] = new[4 + j] + t * t
        return tuple(new)

    z_copy(0, z0, sz0).start()
    i_copy(0, i0, si0).wait()
    g_copy(i0, rows0, sg0).start()

    zero = jnp.zeros((L,), jnp.float32)

    def body(s, accs):
        c0 = 2 * s
        c1 = c0 + 1
        i_copy(c1, i1, si1).wait()
        g_copy(i1, rows1, sg1).start()
        z_copy(c1, z1, sz1).start()

        g_copy(i0, rows0, sg0).wait()

        @pl.when(s + 1 < NH)
        def _():
            i_copy(c0 + 2, i0, si0).start()

        z_copy(c0, z0, sz0).wait()

        @pl.when(s > 0)
        def _():
            w_copy(c0 - 2, t0, sw0).wait()

        accs = compute(rows0, z0, t0, accs)
        w_copy(c0, t0, sw0).start()

        g_copy(i1, rows1, sg1).wait()

        @pl.when(s + 1 < NH)
        def _():
            i_copy(c1 + 2, i1, si1).start()
            i_copy(c0 + 2, i0, si0).wait()
            g_copy(i0, rows0, sg0).start()
            z_copy(c0 + 2, z0, sz0).start()

        z_copy(c1, z1, sz1).wait()

        @pl.when(s > 0)
        def _():
            w_copy(c1 - 2, t1, sw1).wait()

        accs = compute(rows1, z1, t1, accs)
        w_copy(c1, t1, sw1).start()
        return accs

    accs = lax.fori_loop(0, NH, body, tuple(zero for _ in range(8)))
    w_copy(NCH - 2, t0, sw0).wait()
    w_copy(NCH - 1, t1, sw1).wait()
    for j in range(D // L):
        accs_v[pl.ds(L * j, L)] = accs[j]
        accq_v[pl.ds(L * j, L)] = accs[4 + j]
    pltpu.sync_copy(accs_v, pss_hbm.at[wid])
    pltpu.sync_copy(accq_v, psq_hbm.at[wid])


def _sc_gather_max(y, z, gidx):
    mesh = plsc.VectorSubcoreMesh(core_axis_name="c", subcore_axis_name="s",
                                  num_cores=2, num_subcores=16)
    f = pl.kernel(
        _sc_gather_max_body,
        out_type=(
            jax.ShapeDtypeStruct((BNT, D), jnp.float32),
            jax.ShapeDtypeStruct((NW, D), jnp.float32),
            jax.ShapeDtypeStruct((NW, D), jnp.float32),
        ),
        mesh=mesh,
        scratch_types=[
            pltpu.VMEM_SHARED((BNT // 2, D), jnp.bfloat16),
            pltpu.VMEM((GK, D), jnp.bfloat16),
            pltpu.VMEM((GK, D), jnp.bfloat16),
            pltpu.VMEM((GK,), jnp.int32),
            pltpu.VMEM((GK,), jnp.int32),
            pltpu.VMEM((G, D), jnp.float32),
            pltpu.VMEM((G, D), jnp.float32),
            pltpu.VMEM((G, D), jnp.float32),
            pltpu.VMEM((G, D), jnp.float32),
            pltpu.VMEM((D,), jnp.float32),
            pltpu.VMEM((D,), jnp.float32),
            pltpu.SemaphoreType.DMA,
            pltpu.SemaphoreType.DMA,
            pltpu.SemaphoreType.DMA,
            pltpu.SemaphoreType.DMA,
            pltpu.SemaphoreType.DMA,
            pltpu.SemaphoreType.DMA,
            pltpu.SemaphoreType.DMA,
            pltpu.SemaphoreType.DMA,
        ],
        compiler_params=pltpu.CompilerParams(use_tc_tiling_on_sc=False,
                                             needs_layout_passes=False),
    )
    return f(y, z, gidx)


_LG = [b2 * 32 + (i // 2) + 16 * (i % 2) for b2 in range(2) for i in range(32)]


def kernel(x, neighbor_ind, W1, W2, gamma1, beta1, gamma2, beta2):
    lg = jnp.array(_LG, dtype=jnp.int32)
    w1y = W1[:, :D].T[:, lg]
    w1z = (W1[:, D:] - W1[:, :D]).T
    w2y = W2[:, :D].T[:, lg]
    w2z = (W2[:, D:] - W2[:, :D]).T
    gidx = (neighbor_ind.astype(jnp.int32)
            + ((jnp.arange(B, dtype=jnp.int32) % (B // 2)) * N)[:, None, None]
            ).reshape(BNT * K)
    g1 = gamma1.reshape(1, D)
    b1 = beta1.reshape(1, D)
    g2 = gamma2.reshape(1, D)
    b2 = beta2.reshape(1, D)

    y1, z1 = _mm_in(x, w1y, w1z)
    t1, ps1, pq1 = _sc_gather_max(y1, z1, gidx)
    y2, z2 = _mm_mid(t1, ps1, pq1, g1, b1, w2y, w2z, x)
    t2, ps2, pq2 = _sc_gather_max(y2, z2, gidx)
    return _mm_out(t2, ps2, pq2, g2, b2)


# speedup vs baseline: 1.4609x; 1.0238x over previous
import functools

import jax
import jax.numpy as jnp
from jax import lax
from jax.experimental import pallas as pl
from jax.experimental.pallas import tpu as pltpu
from jax.experimental.pallas import tpu_sc as plsc

B, D, N, K = 8, 64, 4096, 16
BNT = B * N
BT = 4096
NB = N // BT
NW = 32
P = BNT // NW
G = 16
GK = G * K
NCH = P // G
NH = NCH // 2
L = 16
EPS = 1e-5


def _gelu(v):
    a1, a2, a3, a4, a5 = (0.254829592, -0.284496736, 1.421413741,
                          -1.453152027, 1.061405429)
    p = 0.3275911
    u = v * 0.7071067811865476
    s = jnp.sign(u)
    ua = jnp.abs(u)
    t = 1.0 / (1.0 + p * ua)
    poly = ((((a5 * t + a4) * t + a3) * t + a2) * t + a1) * t
    erf = s * (1.0 - poly * jnp.exp(-ua * ua))
    return 0.5 * v * (1.0 + erf)


def _mm_in_body(x_ref, wy_ref, wz_ref, y_ref, z_ref):
    xb = x_ref[0]
    dn = (((0,), (0,)), ((), ()))
    y_ref[...] = lax.dot_general(
        xb, wy_ref[...], dn,
        preferred_element_type=jnp.float32).astype(jnp.bfloat16)
    z_ref[...] = lax.dot_general(xb, wz_ref[...], dn,
                                 preferred_element_type=jnp.float32)


def _mm_in(x, wy, wz):
    return pl.pallas_call(
        _mm_in_body,
        grid=(B, NB),
        in_specs=[
            pl.BlockSpec((1, D, BT), lambda b, j: (b, 0, j)),
            pl.BlockSpec((D, D), lambda b, j: (0, 0)),
            pl.BlockSpec((D, D), lambda b, j: (0, 0)),
        ],
        out_specs=[pl.BlockSpec((BT, D), lambda b, j: (b * NB + j, 0))] * 2,
        out_shape=[jax.ShapeDtypeStruct((BNT, D), jnp.bfloat16),
                   jax.ShapeDtypeStruct((BNT, D), jnp.float32)],
    )(x, wy, wz)


def _bn_coeffs(ps, pq, g, bt):
    ssum = jnp.sum(ps, axis=0)
    ssq = jnp.sum(pq, axis=0)
    mean = ssum * (1.0 / BNT)
    var = ssq * (1.0 / BNT) - mean * mean
    scale = g[0] * lax.rsqrt(var + EPS)
    shift = bt[0] - mean * scale
    return scale, shift


def _mm_mid_body(t_ref, ps_ref, pq_ref, g_ref, b_ref, wy_ref, wz_ref, x_ref,
                 y_ref, z_ref):
    scale, shift = _bn_coeffs(ps_ref[...], pq_ref[...], g_ref[...], b_ref[...])
    h = _gelu(t_ref[...] * scale[None, :] + shift[None, :])
    dn = (((1,), (0,)), ((), ()))
    y_ref[...] = lax.dot_general(
        h, wy_ref[...], dn,
        preferred_element_type=jnp.float32).astype(jnp.bfloat16)
    z_ref[...] = lax.dot_general(h, wz_ref[...], dn,
                                 preferred_element_type=jnp.float32) \
        + jnp.transpose(x_ref[0])


def _mm_mid(t1, ps, pq, g, bt, wy, wz, x):
    return pl.pallas_call(
        _mm_mid_body,
        grid=(B, NB),
        in_specs=[
            pl.BlockSpec((BT, D), lambda b, j: (b * NB + j, 0)),
            pl.BlockSpec((NW, D), lambda b, j: (0, 0)),
            pl.BlockSpec((NW, D), lambda b, j: (0, 0)),
            pl.BlockSpec((1, D), lambda b, j: (0, 0)),
            pl.BlockSpec((1, D), lambda b, j: (0, 0)),
            pl.BlockSpec((D, D), lambda b, j: (0, 0)),
            pl.BlockSpec((D, D), lambda b, j: (0, 0)),
            pl.BlockSpec((1, D, BT), lambda b, j: (b, 0, j)),
        ],
        out_specs=[pl.BlockSpec((BT, D), lambda b, j: (b * NB + j, 0))] * 2,
        out_shape=[jax.ShapeDtypeStruct((BNT, D), jnp.bfloat16),
                   jax.ShapeDtypeStruct((BNT, D), jnp.float32)],
    )(t1, ps, pq, g, bt, wy, wz, x)


def _mm_out_body(t_ref, ps_ref, pq_ref, g_ref, b_ref, out_ref):
    scale, shift = _bn_coeffs(ps_ref[...], pq_ref[...], g_ref[...], b_ref[...])
    r = _gelu(t_ref[...] * scale[None, :] + shift[None, :])
    out_ref[0] = jnp.transpose(r)


def _mm_out(t2, ps, pq, g, bt):
    return pl.pallas_call(
        _mm_out_body,
        grid=(B, NB),
        in_specs=[
            pl.BlockSpec((BT, D), lambda b, j: (b * NB + j, 0)),
            pl.BlockSpec((NW, D), lambda b, j: (0, 0)),
            pl.BlockSpec((NW, D), lambda b, j: (0, 0)),
            pl.BlockSpec((1, D), lambda b, j: (0, 0)),
            pl.BlockSpec((1, D), lambda b, j: (0, 0)),
        ],
        out_specs=pl.BlockSpec((1, D, BT), lambda b, j: (b, 0, j)),
        out_shape=jax.ShapeDtypeStruct((B, D, N), jnp.float32),
    )(t2, ps, pq, g, bt)


def _sc_gather_max_body(y_hbm, z_hbm, gidx_hbm, t_hbm, pss_hbm, psq_hbm,
                        ysh, rows0, rows1, rows2, i0, i1, i2,
                        z0, z1, z2, t0, t1, t2, accs_v, accq_v,
                        sg0, sg1, sg2, si0, si1, si2,
                        sz0, sz1, sz2, sw0, sw1, sw2):
    cid = lax.axis_index("c")
    sid = lax.axis_index("s")
    wid = cid * 16 + sid
    base = wid * P

    def i_copy(c, i_v, sem):
        return pltpu.make_async_copy(
            gidx_hbm.at[pl.ds((base + c * G) * K, GK)], i_v, sem)

    i_copy(0, i0, si0).start()
    i_copy(1, i1, si1).start()

    hb = BNT // 2
    st = hb // 16
    pltpu.sync_copy(y_hbm.at[pl.ds(cid * hb + sid * st, st)],
                    ysh.at[pl.ds(sid * st, st)])
    plsc.subcore_barrier()

    def g_copy(i_v, rows_v, sem):
        return pltpu.make_async_copy(ysh.at[i_v], rows_v, sem)

    def z_copy(c, z_v, sem):
        return pltpu.make_async_copy(z_hbm.at[pl.ds(base + c * G, G)], z_v, sem)

    def w_copy(c, t_v, sem):
        return pltpu.make_async_copy(t_v, t_hbm.at[pl.ds(base + c * G, G)], sem)

    def compute(rows_v, z_v, t_v, accs):
        new = list(accs)
        for i in range(G):
            for j2 in range(D // (2 * L)):
                sl = pl.ds(2 * L * j2, 2 * L)
                m = rows_v[i * K, sl]
                for kk in range(1, K):
                    m = jnp.maximum(m, rows_v[i * K + kk, sl])
                ga, gb = plsc.unpack(m, format=plsc.PackFormat.INTERLEAVED)
                for j, gv in ((2 * j2, ga), (2 * j2 + 1, gb)):
                    sj = pl.ds(L * j, L)
                    t = gv + z_v[i, sj]
                    t_v[i, sj] = t
                    new[j] = new[j] + t
                    new[4 + j] = new[4 + j] + t * t
        return tuple(new)

    bufs = ((rows0, i0, z0, t0, sg0, si0, sz0, sw0),
            (rows1, i1, z1, t1, sg1, si1, sz1, sw1),
            (rows2, i2, z2, t2, sg2, si2, sz2, sw2))

    i_copy(2, i2, si2).start()
    z_copy(0, z0, sz0).start()
    z_copy(1, z1, sz1).start()
    i_copy(0, i0, si0).wait()
    g_copy(i0, rows0, sg0).start()
    i_copy(1, i1, si1).wait()
    g_copy(i1, rows1, sg1).start()

    zero = jnp.zeros((L,), jnp.float32)

    def section(c, x, accs):
        rx, ix, zx, tx, sgx, six, szx, swx = bufs[x]
        r2_, i2_, z2_, t2_, sg2_, si2_, sz2_, sw2_ = bufs[(x + 2) % 3]
        g_copy(ix, rx, sgx).wait()

        @pl.when(c + 3 < NCH)
        def _():
            i_copy(c + 3, ix, six).start()

        @pl.when(c + 2 < NCH)
        def _():
            i_copy(c + 2, i2_, si2_).wait()
            g_copy(i2_, r2_, sg2_).start()
            z_copy(c + 2, z2_, sz2_).start()

        z_copy(c, zx, szx).wait()

        @pl.when(c >= 3)
        def _():
            w_copy(c - 3, tx, swx).wait()

        accs = compute(rx, zx, tx, accs)
        w_copy(c, tx, swx).start()
        return accs

    def body(s, accs):
        c0 = 3 * s
        accs = section(c0, 0, accs)
        accs = section(c0 + 1, 1, accs)
        accs = section(c0 + 2, 2, accs)
        return accs

    accs = lax.fori_loop(0, NCH // 3, body, tuple(zero for _ in range(8)))
    for c in range((NCH // 3) * 3, NCH):
        accs = section(jnp.int32(c), c % 3, accs)
    w_copy(NCH - 3, bufs[(NCH - 3) % 3][3], bufs[(NCH - 3) % 3][7]).wait()
    w_copy(NCH - 2, bufs[(NCH - 2) % 3][3], bufs[(NCH - 2) % 3][7]).wait()
    w_copy(NCH - 1, bufs[(NCH - 1) % 3][3], bufs[(NCH - 1) % 3][7]).wait()
    for j in range(D // L):
        accs_v[pl.ds(L * j, L)] = accs[j]
        accq_v[pl.ds(L * j, L)] = accs[4 + j]
    pltpu.sync_copy(accs_v, pss_hbm.at[wid])
    pltpu.sync_copy(accq_v, psq_hbm.at[wid])


def _sc_gather_max(y, z, gidx):
    mesh = plsc.VectorSubcoreMesh(core_axis_name="c", subcore_axis_name="s",
                                  num_cores=2, num_subcores=16)
    f = pl.kernel(
        _sc_gather_max_body,
        out_type=(
            jax.ShapeDtypeStruct((BNT, D), jnp.float32),
            jax.ShapeDtypeStruct((NW, D), jnp.float32),
            jax.ShapeDtypeStruct((NW, D), jnp.float32),
        ),
        mesh=mesh,
        scratch_types=(
            [pltpu.VMEM_SHARED((BNT // 2, D), jnp.bfloat16)]
            + [pltpu.VMEM((GK, D), jnp.bfloat16)] * 3
            + [pltpu.VMEM((GK,), jnp.int32)] * 3
            + [pltpu.VMEM((G, D), jnp.float32)] * 6
            + [pltpu.VMEM((D,), jnp.float32)] * 2
            + [pltpu.SemaphoreType.DMA] * 12
        ),
        compiler_params=pltpu.CompilerParams(use_tc_tiling_on_sc=False,
                                             needs_layout_passes=False),
    )
    return f(y, z, gidx)


_LG = [b2 * 32 + (i // 2) + 16 * (i % 2) for b2 in range(2) for i in range(32)]


def kernel(x, neighbor_ind, W1, W2, gamma1, beta1, gamma2, beta2):
    lg = jnp.array(_LG, dtype=jnp.int32)
    w1y = W1[:, :D].T[:, lg]
    w1z = (W1[:, D:] - W1[:, :D]).T
    w2y = W2[:, :D].T[:, lg]
    w2z = (W2[:, D:] - W2[:, :D]).T
    gidx = (neighbor_ind.astype(jnp.int32)
            + ((jnp.arange(B, dtype=jnp.int32) % (B // 2)) * N)[:, None, None]
            ).reshape(BNT * K)
    g1 = gamma1.reshape(1, D)
    b1 = beta1.reshape(1, D)
    g2 = gamma2.reshape(1, D)
    b2 = beta2.reshape(1, D)

    y1, z1 = _mm_in(x, w1y, w1z)
    t1, ps1, pq1 = _sc_gather_max(y1, z1, gidx)
    y2, z2 = _mm_mid(t1, ps1, pq1, g1, b1, w2y, w2z, x)
    t2, ps2, pq2 = _sc_gather_max(y2, z2, gidx)
    return _mm_out(t2, ps2, pq2, g2, b2)


# speedup vs baseline: 1.5080x; 1.0323x over previous
import functools

import jax
import jax.numpy as jnp
from jax import lax
from jax.experimental import pallas as pl
from jax.experimental.pallas import tpu as pltpu
from jax.experimental.pallas import tpu_sc as plsc

B, D, N, K = 8, 64, 4096, 16
BNT = B * N
BT = 4096
NB = N // BT
NW = 32
P = BNT // NW
G = 16
GK = G * K
NCH = P // G
NH = NCH // 2
L = 16
EPS = 1e-5


def _gelu(v):
    c = 0.7978845608028654
    return 0.5 * v * (1.0 + jnp.tanh(c * (v + 0.044715 * v * v * v)))


def _mm_in_body(x_ref, wy_ref, wz_ref, y_ref, z_ref):
    xb = x_ref[0]
    dn = (((0,), (0,)), ((), ()))
    y_ref[...] = lax.dot_general(
        xb, wy_ref[...], dn,
        preferred_element_type=jnp.float32).astype(jnp.bfloat16)
    z_ref[...] = lax.dot_general(xb, wz_ref[...], dn,
                                 preferred_element_type=jnp.float32)


def _mm_in(x, wy, wz):
    return pl.pallas_call(
        _mm_in_body,
        grid=(B, NB),
        in_specs=[
            pl.BlockSpec((1, D, BT), lambda b, j: (b, 0, j)),
            pl.BlockSpec((D, D), lambda b, j: (0, 0)),
            pl.BlockSpec((D, D), lambda b, j: (0, 0)),
        ],
        out_specs=[pl.BlockSpec((BT, D), lambda b, j: (b * NB + j, 0))] * 2,
        out_shape=[jax.ShapeDtypeStruct((BNT, D), jnp.bfloat16),
                   jax.ShapeDtypeStruct((BNT, D), jnp.float32)],
    )(x, wy, wz)


def _bn_coeffs(ps, pq, g, bt):
    ssum = jnp.sum(ps, axis=0)
    ssq = jnp.sum(pq, axis=0)
    mean = ssum * (1.0 / BNT)
    var = ssq * (1.0 / BNT) - mean * mean
    scale = g[0] * lax.rsqrt(var + EPS)
    shift = bt[0] - mean * scale
    return scale, shift


def _mm_mid_body(t_ref, ps_ref, pq_ref, g_ref, b_ref, wy_ref, wz_ref, x_ref,
                 y_ref, z_ref):
    scale, shift = _bn_coeffs(ps_ref[...], pq_ref[...], g_ref[...], b_ref[...])
    h = _gelu(t_ref[...] * scale[None, :] + shift[None, :])
    dn = (((1,), (0,)), ((), ()))
    y_ref[...] = lax.dot_general(
        h, wy_ref[...], dn,
        preferred_element_type=jnp.float32).astype(jnp.bfloat16)
    z_ref[...] = lax.dot_general(h, wz_ref[...], dn,
                                 preferred_element_type=jnp.float32) \
        + jnp.transpose(x_ref[0])


def _mm_mid(t1, ps, pq, g, bt, wy, wz, x):
    return pl.pallas_call(
        _mm_mid_body,
        grid=(B, NB),
        in_specs=[
            pl.BlockSpec((BT, D), lambda b, j: (b * NB + j, 0)),
            pl.BlockSpec((NW, D), lambda b, j: (0, 0)),
            pl.BlockSpec((NW, D), lambda b, j: (0, 0)),
            pl.BlockSpec((1, D), lambda b, j: (0, 0)),
            pl.BlockSpec((1, D), lambda b, j: (0, 0)),
            pl.BlockSpec((D, D), lambda b, j: (0, 0)),
            pl.BlockSpec((D, D), lambda b, j: (0, 0)),
            pl.BlockSpec((1, D, BT), lambda b, j: (b, 0, j)),
        ],
        out_specs=[pl.BlockSpec((BT, D), lambda b, j: (b * NB + j, 0))] * 2,
        out_shape=[jax.ShapeDtypeStruct((BNT, D), jnp.bfloat16),
                   jax.ShapeDtypeStruct((BNT, D), jnp.float32)],
    )(t1, ps, pq, g, bt, wy, wz, x)


def _mm_out_body(t_ref, ps_ref, pq_ref, g_ref, b_ref, out_ref):
    scale, shift = _bn_coeffs(ps_ref[...], pq_ref[...], g_ref[...], b_ref[...])
    r = _gelu(t_ref[...] * scale[None, :] + shift[None, :])
    out_ref[0] = jnp.transpose(r)


def _mm_out(t2, ps, pq, g, bt):
    return pl.pallas_call(
        _mm_out_body,
        grid=(B, NB),
        in_specs=[
            pl.BlockSpec((BT, D), lambda b, j: (b * NB + j, 0)),
            pl.BlockSpec((NW, D), lambda b, j: (0, 0)),
            pl.BlockSpec((NW, D), lambda b, j: (0, 0)),
            pl.BlockSpec((1, D), lambda b, j: (0, 0)),
            pl.BlockSpec((1, D), lambda b, j: (0, 0)),
        ],
        out_specs=pl.BlockSpec((1, D, BT), lambda b, j: (b, 0, j)),
        out_shape=jax.ShapeDtypeStruct((B, D, N), jnp.float32),
    )(t2, ps, pq, g, bt)


def _sc_gather_max_body(y_hbm, z_hbm, gidx_hbm, t_hbm, pss_hbm, psq_hbm,
                        ysh, rows0, rows1, rows2, i0, i1, i2,
                        z0, z1, z2, t0, t1, t2, accs_v, accq_v,
                        sg0, sg1, sg2, si0, si1, si2,
                        sz0, sz1, sz2, sw0, sw1, sw2):
    cid = lax.axis_index("c")
    sid = lax.axis_index("s")
    wid = cid * 16 + sid
    base = wid * P

    def i_copy(c, i_v, sem):
        return pltpu.make_async_copy(
            gidx_hbm.at[pl.ds((base + c * G) * K, GK)], i_v, sem)

    i_copy(0, i0, si0).start()
    i_copy(1, i1, si1).start()

    hb = BNT // 2
    st = hb // 16
    pltpu.sync_copy(y_hbm.at[pl.ds(cid * hb + sid * st, st)],
                    ysh.at[pl.ds(sid * st, st)])
    plsc.subcore_barrier()

    def g_copy(i_v, rows_v, sem):
        return pltpu.make_async_copy(ysh.at[i_v], rows_v, sem)

    def z_copy(c, z_v, sem):
        return pltpu.make_async_copy(z_hbm.at[pl.ds(base + c * G, G)], z_v, sem)

    def w_copy(c, t_v, sem):
        return pltpu.make_async_copy(t_v, t_hbm.at[pl.ds(base + c * G, G)], sem)

    def compute(rows_v, z_v, t_v, accs):
        new = list(accs)
        for i in range(G):
            for j2 in range(D // (2 * L)):
                sl = pl.ds(2 * L * j2, 2 * L)
                m = rows_v[i * K, sl]
                for kk in range(1, K):
                    m = jnp.maximum(m, rows_v[i * K + kk, sl])
                ga, gb = plsc.unpack(m, format=plsc.PackFormat.INTERLEAVED)
                for j, gv in ((2 * j2, ga), (2 * j2 + 1, gb)):
                    sj = pl.ds(L * j, L)
                    t = gv + z_v[i, sj]
                    t_v[i, sj] = t
                    new[j] = new[j] + t
                    new[4 + j] = new[4 + j] + t * t
        return tuple(new)

    bufs = ((rows0, i0, z0, t0, sg0, si0, sz0, sw0),
            (rows1, i1, z1, t1, sg1, si1, sz1, sw1),
            (rows2, i2, z2, t2, sg2, si2, sz2, sw2))

    i_copy(2, i2, si2).start()
    z_copy(0, z0, sz0).start()
    z_copy(1, z1, sz1).start()
    i_copy(0, i0, si0).wait()
    g_copy(i0, rows0, sg0).start()
    i_copy(1, i1, si1).wait()
    g_copy(i1, rows1, sg1).start()

    zero = jnp.zeros((L,), jnp.float32)

    def section(c, x, accs):
        rx, ix, zx, tx, sgx, six, szx, swx = bufs[x]
        r2_, i2_, z2_, t2_, sg2_, si2_, sz2_, sw2_ = bufs[(x + 2) % 3]
        g_copy(ix, rx, sgx).wait()

        @pl.when(c + 3 < NCH)
        def _():
            i_copy(c + 3, ix, six).start()

        @pl.when(c + 2 < NCH)
        def _():
            i_copy(c + 2, i2_, si2_).wait()
            g_copy(i2_, r2_, sg2_).start()
            z_copy(c + 2, z2_, sz2_).start()

        z_copy(c, zx, szx).wait()

        @pl.when(c >= 3)
        def _():
            w_copy(c - 3, tx, swx).wait()

        accs = compute(rx, zx, tx, accs)
        w_copy(c, tx, swx).start()
        return accs

    def body(s, accs):
        c0 = 3 * s
        accs = section(c0, 0, accs)
        accs = section(c0 + 1, 1, accs)
        accs = section(c0 + 2, 2, accs)
        return accs

    accs = lax.fori_loop(0, NCH // 3, body, tuple(zero for _ in range(8)))
    for c in range((NCH // 3) * 3, NCH):
        accs = section(jnp.int32(c), c % 3, accs)
    w_copy(NCH - 3, bufs[(NCH - 3) % 3][3], bufs[(NCH - 3) % 3][7]).wait()
    w_copy(NCH - 2, bufs[(NCH - 2) % 3][3], bufs[(NCH - 2) % 3][7]).wait()
    w_copy(NCH - 1, bufs[(NCH - 1) % 3][3], bufs[(NCH - 1) % 3][7]).wait()
    for j in range(D // L):
        accs_v[pl.ds(L * j, L)] = accs[j]
        accq_v[pl.ds(L * j, L)] = accs[4 + j]
    pltpu.sync_copy(accs_v, pss_hbm.at[wid])
    pltpu.sync_copy(accq_v, psq_hbm.at[wid])


def _sc_gather_max(y, z, gidx):
    mesh = plsc.VectorSubcoreMesh(core_axis_name="c", subcore_axis_name="s",
                                  num_cores=2, num_subcores=16)
    f = pl.kernel(
        _sc_gather_max_body,
        out_type=(
            jax.ShapeDtypeStruct((BNT, D), jnp.float32),
            jax.ShapeDtypeStruct((NW, D), jnp.float32),
            jax.ShapeDtypeStruct((NW, D), jnp.float32),
        ),
        mesh=mesh,
        scratch_types=(
            [pltpu.VMEM_SHARED((BNT // 2, D), jnp.bfloat16)]
            + [pltpu.VMEM((GK, D), jnp.bfloat16)] * 3
            + [pltpu.VMEM((GK,), jnp.int32)] * 3
            + [pltpu.VMEM((G, D), jnp.float32)] * 6
            + [pltpu.VMEM((D,), jnp.float32)] * 2
            + [pltpu.SemaphoreType.DMA] * 12
        ),
        compiler_params=pltpu.CompilerParams(use_tc_tiling_on_sc=False,
                                             needs_layout_passes=False),
    )
    return f(y, z, gidx)


_LG = [b2 * 32 + (i // 2) + 16 * (i % 2) for b2 in range(2) for i in range(32)]


def kernel(x, neighbor_ind, W1, W2, gamma1, beta1, gamma2, beta2):
    lg = jnp.array(_LG, dtype=jnp.int32)
    w1y = W1[:, :D].T[:, lg]
    w1z = (W1[:, D:] - W1[:, :D]).T
    w2y = W2[:, :D].T[:, lg]
    w2z = (W2[:, D:] - W2[:, :D]).T
    gidx = (neighbor_ind.astype(jnp.int32)
            + ((jnp.arange(B, dtype=jnp.int32) % (B // 2)) * N)[:, None, None]
            ).reshape(BNT * K)
    g1 = gamma1.reshape(1, D)
    b1 = beta1.reshape(1, D)
    g2 = gamma2.reshape(1, D)
    b2 = beta2.reshape(1, D)

    y1, z1 = _mm_in(x, w1y, w1z)
    t1, ps1, pq1 = _sc_gather_max(y1, z1, gidx)
    y2, z2 = _mm_mid(t1, ps1, pq1, g1, b1, w2y, w2z, x)
    t2, ps2, pq2 = _sc_gather_max(y2, z2, gidx)
    return _mm_out(t2, ps2, pq2, g2, b2)


# speedup vs baseline: 1.5089x; 1.0006x over previous
import functools

import jax
import jax.numpy as jnp
from jax import lax
from jax.experimental import pallas as pl
from jax.experimental.pallas import tpu as pltpu
from jax.experimental.pallas import tpu_sc as plsc

B, D, N, K = 8, 64, 4096, 16
BNT = B * N
BT = 4096
NB = N // BT
NW = 32
P = BNT // NW
G = 16
GK = G * K
NCH = P // G
L = 16
EPS = 1e-5


def _gelu(v):
    c = 0.7978845608028654
    return 0.5 * v * (1.0 + jnp.tanh(c * (v + 0.044715 * v * v * v)))


def _mm_in_body(x_ref, wy_ref, wz_ref, y_ref, z_ref):
    xb = x_ref[0]
    dn = (((0,), (0,)), ((), ()))
    y_ref[...] = lax.dot_general(
        xb, wy_ref[...], dn,
        preferred_element_type=jnp.float32).astype(jnp.bfloat16)
    z_ref[...] = lax.dot_general(xb, wz_ref[...], dn,
                                 preferred_element_type=jnp.float32)


def _mm_in(x, wy, wz):
    return pl.pallas_call(
        _mm_in_body,
        grid=(B, NB),
        in_specs=[
            pl.BlockSpec((1, D, BT), lambda b, j: (b, 0, j)),
            pl.BlockSpec((D, D), lambda b, j: (0, 0)),
            pl.BlockSpec((D, D), lambda b, j: (0, 0)),
        ],
        out_specs=[pl.BlockSpec((BT, D), lambda b, j: (b * NB + j, 0))] * 2,
        out_shape=[jax.ShapeDtypeStruct((BNT, D), jnp.bfloat16),
                   jax.ShapeDtypeStruct((BNT, D), jnp.float32)],
    )(x, wy, wz)


def _bn_coeffs(ps, pq, g, bt):
    ssum = jnp.sum(ps, axis=0)
    ssq = jnp.sum(pq, axis=0)
    mean = ssum * (1.0 / BNT)
    var = ssq * (1.0 / BNT) - mean * mean
    scale = g[0] * lax.rsqrt(var + EPS)
    shift = bt[0] - mean * scale
    return scale, shift


def _mm_mid_body(t_ref, ps_ref, pq_ref, g_ref, b_ref, wy_ref, wz_ref, x_ref,
                 y_ref, z_ref):
    scale, shift = _bn_coeffs(ps_ref[...], pq_ref[...], g_ref[...], b_ref[...])
    h = _gelu(t_ref[...] * scale[None, :] + shift[None, :])
    dn = (((1,), (0,)), ((), ()))
    y_ref[...] = lax.dot_general(
        h, wy_ref[...], dn,
        preferred_element_type=jnp.float32).astype(jnp.bfloat16)
    z_ref[...] = lax.dot_general(h, wz_ref[...], dn,
                                 preferred_element_type=jnp.float32) \
        + jnp.transpose(x_ref[0])


def _mm_mid(t1, ps, pq, g, bt, wy, wz, x):
    return pl.pallas_call(
        _mm_mid_body,
        grid=(B, NB),
        in_specs=[
            pl.BlockSpec((BT, D), lambda b, j: (b * NB + j, 0)),
            pl.BlockSpec((NW, D), lambda b, j: (0, 0)),
            pl.BlockSpec((NW, D), lambda b, j: (0, 0)),
            pl.BlockSpec((1, D), lambda b, j: (0, 0)),
            pl.BlockSpec((1, D), lambda b, j: (0, 0)),
            pl.BlockSpec((D, D), lambda b, j: (0, 0)),
            pl.BlockSpec((D, D), lambda b, j: (0, 0)),
            pl.BlockSpec((1, D, BT), lambda b, j: (b, 0, j)),
        ],
        out_specs=[pl.BlockSpec((BT, D), lambda b, j: (b * NB + j, 0))] * 2,
        out_shape=[jax.ShapeDtypeStruct((BNT, D), jnp.bfloat16),
                   jax.ShapeDtypeStruct((BNT, D), jnp.float32)],
    )(t1, ps, pq, g, bt, wy, wz, x)


def _mm_out_body(t_ref, ps_ref, pq_ref, g_ref, b_ref, out_ref):
    scale, shift = _bn_coeffs(ps_ref[...], pq_ref[...], g_ref[...], b_ref[...])
    r = _gelu(t_ref[...] * scale[None, :] + shift[None, :])
    out_ref[0] = jnp.transpose(r)


def _mm_out(t2, ps, pq, g, bt):
    return pl.pallas_call(
        _mm_out_body,
        grid=(B, NB),
        in_specs=[
            pl.BlockSpec((BT, D), lambda b, j: (b * NB + j, 0)),
            pl.BlockSpec((NW, D), lambda b, j: (0, 0)),
            pl.BlockSpec((NW, D), lambda b, j: (0, 0)),
            pl.BlockSpec((1, D), lambda b, j: (0, 0)),
            pl.BlockSpec((1, D), lambda b, j: (0, 0)),
        ],
        out_specs=pl.BlockSpec((1, D, BT), lambda b, j: (b, 0, j)),
        out_shape=jax.ShapeDtypeStruct((B, D, N), jnp.float32),
    )(t2, ps, pq, g, bt)


def _sc_gather_max_body(y_hbm, z_hbm, gidx_hbm, t_hbm, pss_hbm, psq_hbm,
                        ysh, rows0, rows1, rows2, i0, i1, i2,
                        z0, z1, z2, t0, t1, t2, accs_v, accq_v,
                        sg0, sg1, sg2, si0, si1, si2,
                        sz0, sz1, sz2, sw0, sw1, sw2):
    cid = lax.axis_index("c")
    sid = lax.axis_index("s")
    wid = cid * 16 + sid
    base = wid * P

    def i_copy(c, i_v, sem):
        return pltpu.make_async_copy(
            gidx_hbm.at[pl.ds((base + c * G) * K, GK)], i_v, sem)

    i_copy(0, i0, si0).start()
    i_copy(1, i1, si1).start()

    hb = BNT // 2
    st = hb // 16
    pltpu.sync_copy(y_hbm.at[pl.ds(cid * hb + sid * st, st)],
                    ysh.at[pl.ds(sid * st, st)])
    plsc.subcore_barrier()

    def g_copy(i_v, rows_v, sem):
        return pltpu.make_async_copy(ysh.at[i_v], rows_v, sem)

    def z_copy(c, z_v, sem):
        return pltpu.make_async_copy(z_hbm.at[pl.ds(base + c * G, G)], z_v, sem)

    def w_copy(c, t_v, sem):
        return pltpu.make_async_copy(t_v, t_hbm.at[pl.ds(base + c * G, G)], sem)

    def compute(rows_v, z_v, t_v, accs):
        new = list(accs)
        for i in range(G):
            for j2 in range(D // (2 * L)):
                sl = pl.ds(2 * L * j2, 2 * L)
                m = rows_v[i * K, sl]
                for kk in range(1, K):
                    m = jnp.maximum(m, rows_v[i * K + kk, sl])
                ga, gb = plsc.unpack(m, format=plsc.PackFormat.INTERLEAVED)
                for j, gv in ((2 * j2, ga), (2 * j2 + 1, gb)):
                    sj = pl.ds(L * j, L)
                    t = gv + z_v[i, sj]
                    t_v[i, sj] = t
                    new[j] = new[j] + t
                    new[4 + j] = new[4 + j] + t * t
        return tuple(new)

    bufs = ((rows0, i0, z0, t0, sg0, si0, sz0, sw0),
            (rows1, i1, z1, t1, sg1, si1, sz1, sw1),
            (rows2, i2, z2, t2, sg2, si2, sz2, sw2))

    i_copy(2, i2, si2).start()
    z_copy(0, z0, sz0).start()
    z_copy(1, z1, sz1).start()
    i_copy(0, i0, si0).wait()
    g_copy(i0, rows0, sg0).start()
    i_copy(1, i1, si1).wait()
    g_copy(i1, rows1, sg1).start()

    zero = jnp.zeros((L,), jnp.float32)

    def section(c, x, accs):
        rx, ix, zx, tx, sgx, six, szx, swx = bufs[x]
        r2_, i2_, z2_, t2_, sg2_, si2_, sz2_, sw2_ = bufs[(x + 2) % 3]
        g_copy(ix, rx, sgx).wait()

        @pl.when(c + 3 < NCH)
        def _():
            i_copy(c + 3, ix, six).start()

        @pl.when(c + 2 < NCH)
        def _():
            i_copy(c + 2, i2_, si2_).wait()
            g_copy(i2_, r2_, sg2_).start()
            z_copy(c + 2, z2_, sz2_).start()

        z_copy(c, zx, szx).wait()

        @pl.when(c >= 3)
        def _():
            w_copy(c - 3, tx, swx).wait()

        accs = compute(rx, zx, tx, accs)
        w_copy(c, tx, swx).start()
        return accs

    def body(s, accs):
        c0 = 3 * s
        accs = section(c0, 0, accs)
        accs = section(c0 + 1, 1, accs)
        accs = section(c0 + 2, 2, accs)
        return accs

    accs = lax.fori_loop(0, NCH // 3, body, tuple(zero for _ in range(8)))
    for c in range((NCH // 3) * 3, NCH):
        accs = section(jnp.int32(c), c % 3, accs)
    w_copy(NCH - 3, bufs[(NCH - 3) % 3][3], bufs[(NCH - 3) % 3][7]).wait()
    w_copy(NCH - 2, bufs[(NCH - 2) % 3][3], bufs[(NCH - 2) % 3][7]).wait()
    w_copy(NCH - 1, bufs[(NCH - 1) % 3][3], bufs[(NCH - 1) % 3][7]).wait()
    for j in range(D // L):
        accs_v[pl.ds(L * j, L)] = accs[j]
        accq_v[pl.ds(L * j, L)] = accs[4 + j]
    pltpu.sync_copy(accs_v, pss_hbm.at[wid])
    pltpu.sync_copy(accq_v, psq_hbm.at[wid])


def _sc_gather_max(y, z, gidx):
    mesh = plsc.VectorSubcoreMesh(core_axis_name="c", subcore_axis_name="s",
                                  num_cores=2, num_subcores=16)
    f = pl.kernel(
        _sc_gather_max_body,
        out_type=(
            jax.ShapeDtypeStruct((BNT, D), jnp.float32),
            jax.ShapeDtypeStruct((NW, D), jnp.float32),
            jax.ShapeDtypeStruct((NW, D), jnp.float32),
        ),
        mesh=mesh,
        scratch_types=(
            [pltpu.VMEM_SHARED((BNT // 2, D), jnp.bfloat16)]
            + [pltpu.VMEM((GK, D), jnp.bfloat16)] * 3
            + [pltpu.VMEM((GK,), jnp.int32)] * 3
            + [pltpu.VMEM((G, D), jnp.float32)] * 6
            + [pltpu.VMEM((D,), jnp.float32)] * 2
            + [pltpu.SemaphoreType.DMA] * 12
        ),
        compiler_params=pltpu.CompilerParams(use_tc_tiling_on_sc=False,
                                             needs_layout_passes=False),
    )
    return f(y, z, gidx)


_LG = [b2 * 32 + (i // 2) + 16 * (i % 2) for b2 in range(2) for i in range(32)]


def kernel(x, neighbor_ind, W1, W2, gamma1, beta1, gamma2, beta2):
    lg = jnp.array(_LG, dtype=jnp.int32)
    w1y = W1[:, :D].T[:, lg]
    w1z = (W1[:, D:] - W1[:, :D]).T
    w2y = W2[:, :D].T[:, lg]
    w2z = (W2[:, D:] - W2[:, :D]).T
    gidx = (neighbor_ind.astype(jnp.int32)
            + ((jnp.arange(B, dtype=jnp.int32) % (B // 2)) * N)[:, None, None]
            ).reshape(BNT * K)
    g1 = gamma1.reshape(1, D)
    b1 = beta1.reshape(1, D)
    g2 = gamma2.reshape(1, D)
    b2 = beta2.reshape(1, D)

    y1, z1 = _mm_in(x, w1y, w1z)
    t1, ps1, pq1 = _sc_gather_max(y1, z1, gidx)
    y2, z2 = _mm_mid(t1, ps1, pq1, g1, b1, w2y, w2z, x)
    t2, ps2, pq2 = _sc_gather_max(y2, z2, gidx)
    return _mm_out(t2, ps2, pq2, g2, b2)
